# baseline (speedup 1.0000x reference)
import jax
import jax.numpy as jnp
from jax import lax
from jax.experimental import pallas as pl
from jax.experimental.pallas import tpu as pltpu
from jax.experimental.pallas import tpu_sc as plsc

H = 128
HH = 64
N = 50000
E_N = 300000
EL_N = 200000
EPAD = 327680
ELPAD = 204800
NPAD = 50176
FCH = 3712
FAGG = 51968
FTR = FCH // 16
TB = 1000

f32 = jnp.float32
i32 = jnp.int32


def _dot(a, b):
    return jnp.dot(a, b, preferred_element_type=f32)


def _pre1_body(xu, xi, Wsui, aui, Wdui, adui, Wsiu, aiu, Wdiu, adiu,
               Wlu, blu, Wli, bli,
               hsu_o, su_o, diu_o, hsi_o, si_o, dui_o,
               linu_o, lini_o):
    xu_ = xu[:]
    xi_ = xi[:]
    hsu = _dot(xu_, Wsui[:])
    hsu_o[:] = hsu
    su_o[:] = _dot(hsu, aui[:])
    diu_o[:] = _dot(_dot(xu_, Wdiu[:]), adiu[:])
    hsi = _dot(xi_, Wsiu[:])
    hsi_o[:] = hsi
    si_o[:] = _dot(hsi, aiu[:])
    dui_o[:] = _dot(_dot(xi_, Wdui[:]), adui[:])
    linu_o[:] = _dot(xu_, Wlu[:]) + blu[:]
    lini_o[:] = _dot(xi_, Wli[:]) + bli[:]


def _mid_body(aggi, aggu, lini, linu, b1ui, b1iu,
              Ws2ui, as2ui, Wd2ui, ad2ui, Ws2iu, as2iu, Wd2iu, ad2iu,
              hs2u_o, s2u_o, d2iu_o, hs2i_o, s2i_o, d2ui_o):
    hi = jnp.maximum(aggi[:] + b1ui[:] + lini[:], 0.0)
    hu = jnp.maximum(aggu[:] + b1iu[:] + linu[:], 0.0)
    hs2u = _dot(hu, Ws2ui[:])
    hs2u_o[:] = hs2u
    s2u_o[:] = _dot(hs2u, as2ui[:])
    d2iu_o[:] = _dot(_dot(hu, Wd2iu[:]), ad2iu[:])
    hs2i = _dot(hi, Ws2iu[:])
    hs2i_o[:] = hs2i
    s2i_o[:] = _dot(hs2i, as2iu[:])
    d2ui_o[:] = _dot(_dot(hi, Wd2ui[:]), ad2ui[:])


def _decpre_body(aggzu, aggzi, b2iu, b2ui, Wtop, Wbot, b1d, Zu_o, Zi_o):
    Zu_o[:] = _dot(aggzu[:] + b2iu[:], Wtop[:]) + b1d[:]
    Zi_o[:] = _dot(aggzi[:] + b2ui[:], Wbot[:])


def _decpost_body(He, w2, b2, out_o):
    h = jnp.maximum(He[:], 0.0)
    z = _dot(h, w2[:]) + b2[:]
    out_o[:] = jax.nn.sigmoid(z)


def _full(shape):
    return pl.BlockSpec(shape, lambda i: (0, 0))


def _rows(width):
    return pl.BlockSpec((TB, width), lambda i: (i, 0))


def _edge2_body(s0_h, d0_h, row0_h, col0_h,
                s1_h, d1_h, row1_h, col1_h,
                eraw0_h, eraw1_h, den_h,
                s_v, d_v, row_v, col_v, col2d_v, eraw_v, zbuf,
                den_sp0, den_sp1):
    cid = lax.axis_index("c")
    sid = lax.axis_index("s")
    wid = sid * 2 + cid

    def zb(k, c):
        zbuf[pl.ds(k * 16, 16)] = jnp.zeros((16,), f32)
        return c
    lax.fori_loop(0, NPAD // 16 // 16, zb, 0)
    dslc = pl.ds(pl.multiple_of(sid * (NPAD // 16), 8), NPAD // 16)
    pltpu.sync_copy(zbuf, den_sp0.at[dslc])
    pltpu.sync_copy(zbuf, den_sp1.at[dslc])
    plsc.subcore_barrier()

    ebase = wid * (EPAD // 32)
    for q, (s_h, d_h, row_h, col_h, eraw_h, den_sp) in enumerate([
            (s0_h, d0_h, row0_h, col0_h, eraw0_h, den_sp0),
            (s1_h, d1_h, row1_h, col1_h, eraw1_h, den_sp1)]):
        pltpu.sync_copy(s_h, s_v)
        pltpu.sync_copy(d_h, d_v)

        def chbody(ch, c0):
            cbase = pl.multiple_of(ebase + ch * 2048, 2048)
            pltpu.sync_copy(row_h.at[pl.ds(cbase, 2048)], row_v)
            pltpu.sync_copy(col_h.at[pl.ds(cbase, 2048)], col_v)

            def body(g, c):
                sl = pl.ds(g * 16, 16)
                rv = row_v[sl]
                cv = col_v[sl]
                sv = plsc.load_gather(s_v, [rv])
                dv = plsc.load_gather(d_v, [cv])
                a = sv + dv
                a = jnp.where(a > 0, a, 0.2 * a)
                e = jnp.exp(a)
                eid = cbase + g * 16 + lax.iota(i32, 16)
                e = jnp.where(eid < E_N, e, 0.0)
                eraw_v[sl] = e
                col2d_v[g // 8, pl.ds((g % 8) * 16, 16)] = cv
                return c
            lax.fori_loop(0, 128, body, 0)
            pltpu.sync_copy(eraw_v, eraw_h.at[pl.ds(cbase, 2048)])

            def kbody(k, c2):
                pltpu.sync_copy(
                    eraw_v.at[pl.ds(pl.multiple_of(k * 128, 128), 128)],
                    den_sp.at[col2d_v.at[k]], add=True)
                return c2
            lax.fori_loop(0, 16, kbody, 0)
            return c0
        lax.fori_loop(0, 5, chbody, 0)
    plsc.subcore_barrier()
    for q, den_sp in enumerate([den_sp0, den_sp1]):
        doff = pl.multiple_of((cid * 2 + q) * NPAD + sid * (NPAD // 16), 8)
        pltpu.sync_copy(den_sp.at[dslc], zbuf)
        pltpu.sync_copy(zbuf, den_h.at[pl.ds(doff, NPAD // 16)])


def _alpha2_body(col0_h, eraw0_h, col1_h, eraw1_h, den_h,
                 alpha0_h, alpha1_h,
                 den_v, den2_v, col_v, e_v, a_v):
    cid = lax.axis_index("c")
    sid = lax.axis_index("s")
    wid = sid * 2 + cid
    ebase = wid * (EPAD // 32)
    for q, (col_h, eraw_h, alpha_h) in enumerate([
            (col0_h, eraw0_h, alpha0_h), (col1_h, eraw1_h, alpha1_h)]):
        pltpu.sync_copy(den_h.at[pl.ds(q * NPAD, NPAD)], den_v)
        pltpu.sync_copy(den_h.at[pl.ds((2 + q) * NPAD, NPAD)], den2_v)

        def addb(k, c):
            sl = pl.ds(k * 16, 16)
            den_v[sl] = den_v[sl] + den2_v[sl]
            return c
        lax.fori_loop(0, NPAD // 16, addb, 0)

        def chbody(ch, c0):
            cbase = pl.multiple_of(ebase + ch * 2048, 2048)
            pltpu.sync_copy(col_h.at[pl.ds(cbase, 2048)], col_v)
            pltpu.sync_copy(eraw_h.at[pl.ds(cbase, 2048)], e_v)

            def body(g, c):
                sl = pl.ds(g * 16, 16)
                cv = col_v[sl]
                ev = e_v[sl]
                dv = plsc.load_gather(den_v, [cv])
                a_v[sl] = ev / (dv + 1e-16)
                return c
            lax.fori_loop(0, 128, body, 0)
            pltpu.sync_copy(a_v, alpha_h.at[pl.ds(cbase, 2048)])
            return c0
        lax.fori_loop(0, 5, chbody, 0)


def _feat2_body(row0_h, col0_h, alpha0_h, hs0_h,
                row1_h, col1_h, alpha1_h, hs1_h,
                agg0_h, agg1_h,
                row_v, col_v, al_v, comp_row, comp_dst, comp_al,
                idx_row, idx_dst, grows, zbuf, out_sp, sem):
    cid = lax.axis_index("c")
    sid = lax.axis_index("s")

    def zb(r, c):
        for k in range(8):
            zbuf[r, pl.ds(k * 16, 16)] = jnp.zeros((16,), f32)
        return c
    lax.fori_loop(0, 128, zb, 0)

    rb = pl.multiple_of(sid * FTR, 8)
    for q, (row_h, col_h, alpha_h, hs_h, agg_h) in enumerate([
            (row0_h, col0_h, alpha0_h, hs0_h, agg0_h),
            (row1_h, col1_h, alpha1_h, hs1_h, agg1_h)]):

        def tbody(t, c9):
            ck = cid * 7 + t
            lo = pl.multiple_of(ck * FCH, 128)
            hi = lo + FCH

            pltpu.sync_copy(
                zbuf, out_sp.at[pl.ds(pl.multiple_of(rb, 8), 128)])
            pltpu.sync_copy(
                zbuf.at[pl.ds(0, 104)],
                out_sp.at[pl.ds(pl.multiple_of(rb + 128, 8), 104)])
            plsc.subcore_barrier()

            sbase = sid * (EPAD // 16)

            def chbody(ch, ptr):
                cbase = pl.multiple_of(sbase + ch * 2048, 2048)
                pltpu.sync_copy(row_h.at[pl.ds(cbase, 2048)], row_v)
                pltpu.sync_copy(col_h.at[pl.ds(cbase, 2048)], col_v)
                pltpu.sync_copy(alpha_h.at[pl.ds(cbase, 2048)], al_v)

                def sbody(g2, ptr):
                    for g in (g2 * 2, g2 * 2 + 1):
                        sl = pl.ds(g * 16, 16)
                        cv = col_v[sl]
                        rv = row_v[sl]
                        av = al_v[sl]
                        m = (cv >= lo) & (cv < hi)
                        mi = jnp.where(m, 1, 0).astype(i32)
                        psl = pl.ds(ptr, 16)
                        plsc.store_compressed(comp_row.at[psl], rv, mask=m)
                        plsc.store_compressed(comp_dst.at[psl], cv - lo,
                                              mask=m)
                        plsc.store_compressed(comp_al.at[psl], av, mask=m)
                        ptr = ptr + jnp.sum(mi)
                    return ptr
                return lax.fori_loop(0, 64, sbody, ptr)
            ptr = lax.fori_loop(0, 10, chbody, jnp.int32(0))

            cntp = ((ptr + 127) // 128) * 128
            zi16 = jnp.zeros((16,), i32)
            zf16 = jnp.zeros((16,), f32)

            def pbody(i, c):
                idxs = ptr + i * 16 + lax.iota(i32, 16)
                pm = idxs < cntp
                plsc.store_scatter(comp_row, [idxs], zi16, mask=pm)
                plsc.store_scatter(comp_dst, [idxs], zi16, mask=pm)
                plsc.store_scatter(comp_al, [idxs], zf16, mask=pm)
                return c
            lax.fori_loop(0, 8, pbody, 0)

            nb = cntp // 128

            def fbody(b, c):
                off = b * 128

                def cp(i, c2):
                    s16 = pl.ds(off + i * 16, 16)
                    d16 = pl.ds(i * 16, 16)
                    idx_row[d16] = comp_row[s16]
                    idx_dst[d16] = comp_dst[s16]
                    return c2
                lax.fori_loop(0, 8, cp, 0)
                pltpu.async_copy(hs_h.at[idx_row], grows, sem).wait()

                def mul_r(r4, c2):
                    for r2 in range(4):
                        r = r4 * 4 + r2
                        av = plsc.load_gather(
                            comp_al, [jnp.full((16,), off + r, i32)])
                        for k in range(8):
                            sl = pl.ds(k * 16, 16)
                            grows[r, sl] = grows[r, sl] * av
                    return c2
                lax.fori_loop(0, 32, mul_r, 0)
                pltpu.sync_copy(grows, out_sp.at[idx_dst], add=True)
                return c
            lax.fori_loop(0, nb, fbody, 0)
            plsc.subcore_barrier()
            roff = pl.multiple_of(rb, 8)
            pltpu.sync_copy(out_sp.at[pl.ds(roff, 128)], grows)
            pltpu.sync_copy(
                grows, agg_h.at[pl.ds(pl.multiple_of(lo + roff, 8), 128)])
            roff = pl.multiple_of(rb + 128, 8)
            pltpu.sync_copy(out_sp.at[pl.ds(roff, 104)],
                            grows.at[pl.ds(0, 104)])
            pltpu.sync_copy(
                grows.at[pl.ds(0, 104)],
                agg_h.at[pl.ds(pl.multiple_of(lo + roff, 8), 104)])
            plsc.subcore_barrier()
            return c9
        lax.fori_loop(0, 7, tbody, 0)


def _dec_body(rowl_h, coll_h, zu_h, zi_h, he_h, idx_u, idx_i, gu, gi, sem):
    cid = lax.axis_index("c")
    sid = lax.axis_index("s")
    wid = sid * 2 + cid
    base = wid * (ELPAD // 32)

    def body(b, c):
        off = pl.multiple_of(base + b * 128, 128)
        pltpu.sync_copy(rowl_h.at[pl.ds(off, 128)], idx_u)
        pltpu.sync_copy(coll_h.at[pl.ds(off, 128)], idx_i)
        pltpu.async_copy(zu_h.at[idx_u], gu, sem).wait()
        pltpu.async_copy(zi_h.at[idx_i], gi, sem).wait()

        def addr(r4, c2):
            for r2 in range(4):
                r = r4 * 4 + r2
                for k in range(8):
                    sl = pl.ds(k * 16, 16)
                    gu[r, sl] = gu[r, sl] + gi[r, sl]
            return c2
        lax.fori_loop(0, 32, addr, 0)
        pltpu.sync_copy(gu, he_h.at[pl.ds(off, 128)])
        return c
    lax.fori_loop(0, ELPAD // 32 // 128, body, 0)


def _mesh():
    return plsc.VectorSubcoreMesh(core_axis_name="c", subcore_axis_name="s")


_SC_PARAMS = pltpu.CompilerParams(needs_layout_passes=False)


def _edge2_call(s0, d0, row0, col0, s1, d1, row1, col1):
    return pl.kernel(
        _edge2_body,
        out_type=[jax.ShapeDtypeStruct((EPAD,), f32),
                  jax.ShapeDtypeStruct((EPAD,), f32),
                  jax.ShapeDtypeStruct((4 * NPAD,), f32)],
        mesh=_mesh(),
        compiler_params=_SC_PARAMS,
        scratch_types=[
            pltpu.VMEM((N,), f32),
            pltpu.VMEM((N,), f32),
            pltpu.VMEM((2048,), i32),
            pltpu.VMEM((2048,), i32),
            pltpu.VMEM((16, 128), i32),
            pltpu.VMEM((2048,), f32),
            pltpu.VMEM((NPAD // 16,), f32),
            pltpu.VMEM_SHARED((NPAD,), f32),
            pltpu.VMEM_SHARED((NPAD,), f32),
        ],
    )(s0, d0, row0, col0, s1, d1, row1, col1)


def _alpha2_call(col0, eraw0, col1, eraw1, den4):
    return pl.kernel(
        _alpha2_body,
        out_type=[jax.ShapeDtypeStruct((EPAD,), f32),
                  jax.ShapeDtypeStruct((EPAD,), f32)],
        mesh=_mesh(),
        compiler_params=_SC_PARAMS,
        scratch_types=[
            pltpu.VMEM((NPAD,), f32),
            pltpu.VMEM((NPAD,), f32),
            pltpu.VMEM((2048,), i32),
            pltpu.VMEM((2048,), f32),
            pltpu.VMEM((2048,), f32),
        ],
    )(col0, eraw0, col1, eraw1, den4)


def _feat2_call(row0, col0, alpha0, hs0, row1, col1, alpha1, hs1):
    return pl.kernel(
        _feat2_body,
        out_type=[jax.ShapeDtypeStruct((FAGG, H), f32),
                  jax.ShapeDtypeStruct((FAGG, H), f32)],
        mesh=_mesh(),
        compiler_params=_SC_PARAMS,
        scratch_types=[
            pltpu.VMEM((2048,), i32),
            pltpu.VMEM((2048,), i32),
            pltpu.VMEM((2048,), f32),
            pltpu.VMEM((20608,), i32),
            pltpu.VMEM((20608,), i32),
            pltpu.VMEM((20608,), f32),
            pltpu.VMEM((128,), i32),
            pltpu.VMEM((128,), i32),
            pltpu.VMEM((128, H), f32),
            pltpu.VMEM((128, H), f32),
            pltpu.VMEM_SHARED((FCH, H), f32),
            pltpu.SemaphoreType.DMA,
        ],
    )(row0, col0, alpha0, hs0, row1, col1, alpha1, hs1)


def _dec_call(rowl, coll, Zu, Zi):
    return pl.kernel(
        _dec_body,
        out_type=[jax.ShapeDtypeStruct((ELPAD, H), f32)],
        mesh=_mesh(),
        compiler_params=_SC_PARAMS,
        scratch_types=[
            pltpu.VMEM((128,), i32),
            pltpu.VMEM((128,), i32),
            pltpu.VMEM((128, H), f32),
            pltpu.VMEM((128, H), f32),
            pltpu.SemaphoreType.DMA,
        ],
    )(rowl, coll, Zu, Zi)[0]


def _run_layer(s0, d0, hs0, s1, d1, hs1,
               row0, col0, row1, col1):
    eraw0, eraw1, den4 = _edge2_call(s0, d0, row0, col0,
                                     s1, d1, row1, col1)
    alpha0, alpha1 = _alpha2_call(col0, eraw0, col1, eraw1, den4)
    agg0, agg1 = _feat2_call(row0, col0, alpha0, hs0,
                             row1, col1, alpha1, hs1)
    return agg0, agg1, alpha0, alpha1


def kernel(x_user, x_item, edge_index_ui, edge_index_iu, edge_label_index,
           params):
    p = params
    def padE(x):
        return jnp.pad(x.astype(i32), (0, EPAD - E_N), mode='wrap')

    def padL(x):
        return jnp.pad(x.astype(i32), (0, ELPAD - EL_N), mode='wrap')

    row_ui = padE(edge_index_ui[0])
    col_ui = padE(edge_index_ui[1])
    row_iu = padE(edge_index_iu[0])
    col_iu = padE(edge_index_iu[1])
    rowl = padL(edge_label_index[0])
    coll = padL(edge_label_index[1])

    c1ui, c1iu = p['conv1_ui'], p['conv1_iu']
    c2ui, c2iu = p['conv2_ui'], p['conv2_iu']
    lu, li = p['lin1_user'], p['lin1_item']

    nt = N // TB
    v = lambda x: x.reshape(H, 1)
    b = lambda x: x.reshape(1, H)
    sds = jax.ShapeDtypeStruct

    (hsu, su1, diu1, hsi, si1, dui1, linu, lini) = pl.pallas_call(
        _pre1_body,
        grid=(nt,),
        in_specs=[_rows(H), _rows(H)] + [_full((H, H)), _full((H, 1))] * 4
        + [_full((H, H)), _full((1, H))] * 2,
        out_specs=[_rows(H), _rows(1), _rows(1), _rows(H), _rows(1),
                   _rows(1), _rows(H), _rows(H)],
        out_shape=[sds((N, H), f32), sds((N, 1), f32), sds((N, 1), f32),
                   sds((N, H), f32), sds((N, 1), f32), sds((N, 1), f32),
                   sds((N, H), f32), sds((N, H), f32)],
    )(x_user, x_item,
      c1ui['Ws'], v(c1ui['as']), c1ui['Wd'], v(c1ui['ad']),
      c1iu['Ws'], v(c1iu['as']), c1iu['Wd'], v(c1iu['ad']),
      lu['W'], b(lu['b']), li['W'], b(li['b']))

    agg_i1, agg_u1, _, _ = _run_layer(
        su1.reshape(-1), dui1.reshape(-1), hsu,
        si1.reshape(-1), diu1.reshape(-1), hsi,
        row_ui, col_ui, row_iu, col_iu)

    (hs2u, s2u, d2iu, hs2i, s2i, d2ui) = pl.pallas_call(
        _mid_body,
        grid=(nt,),
        in_specs=[_rows(H)] * 4 + [_full((1, H))] * 2
        + [_full((H, H)), _full((H, 1))] * 4,
        out_specs=[_rows(H), _rows(1), _rows(1), _rows(H), _rows(1),
                   _rows(1)],
        out_shape=[sds((N, H), f32), sds((N, 1), f32), sds((N, 1), f32),
                   sds((N, H), f32), sds((N, 1), f32), sds((N, 1), f32)],
    )(agg_i1[:N], agg_u1[:N], lini, linu, b(c1ui['b']), b(c1iu['b']),
      c2ui['Ws'], v(c2ui['as']), c2ui['Wd'], v(c2ui['ad']),
      c2iu['Ws'], v(c2iu['as']), c2iu['Wd'], v(c2iu['ad']))

    agg_zi, agg_zu, alpha_ui, alpha_iu = _run_layer(
        s2u.reshape(-1), d2ui.reshape(-1), hs2u,
        s2i.reshape(-1), d2iu.reshape(-1), hs2i,
        row_ui, col_ui, row_iu, col_iu)

    Wd1 = p['dec1']['W']
    Zu, Zi = pl.pallas_call(
        _decpre_body,
        grid=(nt,),
        in_specs=[_rows(H), _rows(H), _full((1, H)), _full((1, H)),
                  _full((H, H)), _full((H, H)), _full((1, H))],
        out_specs=[_rows(H), _rows(H)],
        out_shape=[sds((N, H), f32), sds((N, H), f32)],
    )(agg_zu[:N], agg_zi[:N], b(c2iu['b']), b(c2ui['b']),
      Wd1[:H], Wd1[H:], b(p['dec1']['b']))

    He = _dec_call(rowl, coll, Zu, Zi)

    predp = pl.pallas_call(
        _decpost_body,
        grid=(ELPAD // 1024,),
        in_specs=[pl.BlockSpec((1024, H), lambda i: (i, 0)),
                  _full((H, 1)), _full((1, 1))],
        out_specs=pl.BlockSpec((1024, 1), lambda i: (i, 0)),
        out_shape=sds((ELPAD, 1), f32),
    )(He, p['dec2']['W'], p['dec2']['b'].reshape(1, 1))

    pred = predp[:EL_N, 0]
    return pred, alpha_ui[:E_N], alpha_iu[:E_N]


# speedup vs baseline: 1.0756x; 1.0756x over previous
import jax
import jax.numpy as jnp
from jax import lax
from jax.experimental import pallas as pl
from jax.experimental.pallas import tpu as pltpu
from jax.experimental.pallas import tpu_sc as plsc

H = 128
HH = 64
N = 50000
E_N = 300000
EL_N = 200000
EPAD = 327680
ELPAD = 204800
NPAD = 50176
FCH = 3712
FAGG = 51968
FTR = FCH // 16
TB = 1000

f32 = jnp.float32
i32 = jnp.int32


def _dot(a, b):
    return jnp.dot(a, b, preferred_element_type=f32)


def _pre1_body(xu, xi, Wsui, aui, Wdui, adui, Wsiu, aiu, Wdiu, adiu,
               Wlu, blu, Wli, bli,
               hsu_o, su_o, diu_o, hsi_o, si_o, dui_o,
               linu_o, lini_o):
    xu_ = xu[:]
    xi_ = xi[:]
    hsu = _dot(xu_, Wsui[:])
    hsu_o[:] = hsu
    su_o[:] = _dot(hsu, aui[:])
    diu_o[:] = _dot(_dot(xu_, Wdiu[:]), adiu[:])
    hsi = _dot(xi_, Wsiu[:])
    hsi_o[:] = hsi
    si_o[:] = _dot(hsi, aiu[:])
    dui_o[:] = _dot(_dot(xi_, Wdui[:]), adui[:])
    linu_o[:] = _dot(xu_, Wlu[:]) + blu[:]
    lini_o[:] = _dot(xi_, Wli[:]) + bli[:]


def _mid_body(aggi, aggu, lini, linu, b1ui, b1iu,
              Ws2ui, as2ui, Wd2ui, ad2ui, Ws2iu, as2iu, Wd2iu, ad2iu,
              hs2u_o, s2u_o, d2iu_o, hs2i_o, s2i_o, d2ui_o):
    hi = jnp.maximum(aggi[:] + b1ui[:] + lini[:], 0.0)
    hu = jnp.maximum(aggu[:] + b1iu[:] + linu[:], 0.0)
    hs2u = _dot(hu, Ws2ui[:])
    hs2u_o[:] = hs2u
    s2u_o[:] = _dot(hs2u, as2ui[:])
    d2iu_o[:] = _dot(_dot(hu, Wd2iu[:]), ad2iu[:])
    hs2i = _dot(hi, Ws2iu[:])
    hs2i_o[:] = hs2i
    s2i_o[:] = _dot(hs2i, as2iu[:])
    d2ui_o[:] = _dot(_dot(hi, Wd2ui[:]), ad2ui[:])


def _decpre_body(aggzu, aggzi, b2iu, b2ui, Wtop, Wbot, b1d, Zu_o, Zi_o):
    Zu_o[:] = _dot(aggzu[:] + b2iu[:], Wtop[:]) + b1d[:]
    Zi_o[:] = _dot(aggzi[:] + b2ui[:], Wbot[:])


def _decpost_body(He, w2, b2, out_o):
    h = jnp.maximum(He[:], 0.0)
    z = _dot(h, w2[:]) + b2[:]
    out_o[:] = jax.nn.sigmoid(z)


def _full(shape):
    return pl.BlockSpec(shape, lambda i: (0, 0))


def _rows(width):
    return pl.BlockSpec((TB, width), lambda i: (i, 0))


def _edge2_body(s0_h, d0_h, row0_h, col0_h,
                s1_h, d1_h, row1_h, col1_h,
                eraw0_h, eraw1_h, den_h,
                s_v, d_v, row_v, col_v, col2d_v, eraw_v, zbuf,
                den_sp0, den_sp1):
    cid = lax.axis_index("c")
    sid = lax.axis_index("s")
    wid = sid * 2 + cid

    def zb(k, c):
        zbuf[pl.ds(k * 16, 16)] = jnp.zeros((16,), f32)
        return c
    lax.fori_loop(0, NPAD // 16 // 16, zb, 0)
    dslc = pl.ds(pl.multiple_of(sid * (NPAD // 16), 8), NPAD // 16)
    pltpu.sync_copy(zbuf, den_sp0.at[dslc])
    pltpu.sync_copy(zbuf, den_sp1.at[dslc])
    plsc.subcore_barrier()

    ebase = wid * (EPAD // 32)
    for q, (s_h, d_h, row_h, col_h, eraw_h, den_sp) in enumerate([
            (s0_h, d0_h, row0_h, col0_h, eraw0_h, den_sp0),
            (s1_h, d1_h, row1_h, col1_h, eraw1_h, den_sp1)]):
        pltpu.sync_copy(s_h, s_v)
        pltpu.sync_copy(d_h, d_v)

        def chbody(ch, c0):
            cbase = pl.multiple_of(ebase + ch * 2048, 2048)
            pltpu.sync_copy(row_h.at[pl.ds(cbase, 2048)], row_v)
            pltpu.sync_copy(col_h.at[pl.ds(cbase, 2048)], col_v)

            def body(g, c):
                sl = pl.ds(g * 16, 16)
                rv = row_v[sl]
                cv = col_v[sl]
                sv = plsc.load_gather(s_v, [rv])
                dv = plsc.load_gather(d_v, [cv])
                a = sv + dv
                a = jnp.where(a > 0, a, 0.2 * a)
                e = jnp.exp(a)
                eid = cbase + g * 16 + lax.iota(i32, 16)
                e = jnp.where(eid < E_N, e, 0.0)
                eraw_v[sl] = e
                col2d_v[g // 8, pl.ds((g % 8) * 16, 16)] = cv
                return c
            lax.fori_loop(0, 128, body, 0)
            pltpu.sync_copy(eraw_v, eraw_h.at[pl.ds(cbase, 2048)])

            def kbody(k, c2):
                pltpu.sync_copy(
                    eraw_v.at[pl.ds(pl.multiple_of(k * 128, 128), 128)],
                    den_sp.at[col2d_v.at[k]], add=True)
                return c2
            lax.fori_loop(0, 16, kbody, 0)
            return c0
        lax.fori_loop(0, 5, chbody, 0)
    plsc.subcore_barrier()
    for q, den_sp in enumerate([den_sp0, den_sp1]):
        doff = pl.multiple_of((cid * 2 + q) * NPAD + sid * (NPAD // 16), 8)
        pltpu.sync_copy(den_sp.at[dslc], zbuf)
        pltpu.sync_copy(zbuf, den_h.at[pl.ds(doff, NPAD // 16)])


def _alpha2_body(col0_h, eraw0_h, col1_h, eraw1_h, den_h,
                 alpha0_h, alpha1_h,
                 den_v, den2_v, col_v, e_v, a_v):
    cid = lax.axis_index("c")
    sid = lax.axis_index("s")
    wid = sid * 2 + cid
    ebase = wid * (EPAD // 32)
    for q, (col_h, eraw_h, alpha_h) in enumerate([
            (col0_h, eraw0_h, alpha0_h), (col1_h, eraw1_h, alpha1_h)]):
        pltpu.sync_copy(den_h.at[pl.ds(q * NPAD, NPAD)], den_v)
        pltpu.sync_copy(den_h.at[pl.ds((2 + q) * NPAD, NPAD)], den2_v)

        def addb(k, c):
            sl = pl.ds(k * 16, 16)
            den_v[sl] = den_v[sl] + den2_v[sl]
            return c
        lax.fori_loop(0, NPAD // 16, addb, 0)

        def chbody(ch, c0):
            cbase = pl.multiple_of(ebase + ch * 2048, 2048)
            pltpu.sync_copy(col_h.at[pl.ds(cbase, 2048)], col_v)
            pltpu.sync_copy(eraw_h.at[pl.ds(cbase, 2048)], e_v)

            def body(g, c):
                sl = pl.ds(g * 16, 16)
                cv = col_v[sl]
                ev = e_v[sl]
                dv = plsc.load_gather(den_v, [cv])
                a_v[sl] = ev / (dv + 1e-16)
                return c
            lax.fori_loop(0, 128, body, 0)
            pltpu.sync_copy(a_v, alpha_h.at[pl.ds(cbase, 2048)])
            return c0
        lax.fori_loop(0, 5, chbody, 0)


def _feat2_body(row0_h, col0_h, alpha0_h, hs0_h,
                row1_h, col1_h, alpha1_h, hs1_h,
                agg0_h, agg1_h,
                row_v, col_v, al_v, comp_pack, comp_al,
                idx_row, idx_dst, idx_row2, idx_dst2, grows, grows2, zbuf,
                out_sp, sem, sem2):
    cid = lax.axis_index("c")
    sid = lax.axis_index("s")

    def zb(r, c):
        for k in range(8):
            zbuf[r, pl.ds(k * 16, 16)] = jnp.zeros((16,), f32)
        return c
    lax.fori_loop(0, 128, zb, 0)

    rb = pl.multiple_of(sid * FTR, 8)
    for q, (row_h, col_h, alpha_h, hs_h, agg_h) in enumerate([
            (row0_h, col0_h, alpha0_h, hs0_h, agg0_h),
            (row1_h, col1_h, alpha1_h, hs1_h, agg1_h)]):

        def tbody(t, c9):
            ck = cid * 7 + t
            lo = pl.multiple_of(ck * FCH, 128)
            hi = lo + FCH

            pltpu.sync_copy(
                zbuf, out_sp.at[pl.ds(pl.multiple_of(rb, 8), 128)])
            pltpu.sync_copy(
                zbuf.at[pl.ds(0, 104)],
                out_sp.at[pl.ds(pl.multiple_of(rb + 128, 8), 104)])
            plsc.subcore_barrier()

            sbase = sid * (EPAD // 16)

            def chbody(ch, ptr):
                cbase = pl.multiple_of(sbase + ch * 2048, 2048)
                pltpu.sync_copy(row_h.at[pl.ds(cbase, 2048)], row_v)
                pltpu.sync_copy(col_h.at[pl.ds(cbase, 2048)], col_v)
                pltpu.sync_copy(alpha_h.at[pl.ds(cbase, 2048)], al_v)

                def sbody(g2, ptr):
                    for g in (g2 * 2, g2 * 2 + 1):
                        sl = pl.ds(g * 16, 16)
                        cv = col_v[sl]
                        rv = row_v[sl]
                        av = al_v[sl]
                        m = (cv >= lo) & (cv < hi)
                        mi = jnp.where(m, 1, 0).astype(i32)
                        pk = rv + ((cv - lo) << 16)
                        psl = pl.ds(ptr, 16)
                        plsc.store_compressed(comp_pack.at[psl], pk, mask=m)
                        plsc.store_compressed(comp_al.at[psl], av, mask=m)
                        ptr = ptr + jnp.sum(mi)
                    return ptr
                return lax.fori_loop(0, 64, sbody, ptr)
            ptr = lax.fori_loop(0, 10, chbody, jnp.int32(0))

            cntp = ((ptr + 127) // 128) * 128
            zi16 = jnp.zeros((16,), i32)
            zf16 = jnp.zeros((16,), f32)

            def pbody(i, c):
                idxs = ptr + i * 16 + lax.iota(i32, 16)
                pm = idxs < cntp
                plsc.store_scatter(comp_pack, [idxs], zi16, mask=pm)
                plsc.store_scatter(comp_al, [idxs], zf16, mask=pm)
                return c
            lax.fori_loop(0, 8, pbody, 0)

            nb = cntp // 128

            def prep(off, idxr, idxd):
                def cp(i, c2):
                    s16 = pl.ds(off + i * 16, 16)
                    d16 = pl.ds(i * 16, 16)
                    pk = comp_pack[s16]
                    idxr[d16] = pk & 0xFFFF
                    idxd[d16] = pk >> 16
                    return c2
                lax.fori_loop(0, 8, cp, 0)

            def mul(off, g):
                def mul_r(r4, c2):
                    for r2 in range(4):
                        r = r4 * 4 + r2
                        av = plsc.load_gather(
                            comp_al, [jnp.full((16,), off + r, i32)])
                        for k in range(8):
                            sl = pl.ds(k * 16, 16)
                            g[r, sl] = g[r, sl] * av
                    return c2
                lax.fori_loop(0, 32, mul_r, 0)

            @pl.when(nb > 0)
            def _():
                prep(0, idx_row, idx_dst)
                pltpu.async_copy(hs_h.at[idx_row], grows, sem)

            def pair(i, c):
                b0 = i * 2
                pltpu.make_async_copy(hs_h.at[idx_row], grows, sem).wait()

                @pl.when(b0 + 1 < nb)
                def _():
                    prep((b0 + 1) * 128, idx_row2, idx_dst2)
                    pltpu.async_copy(hs_h.at[idx_row2], grows2, sem2)
                mul(b0 * 128, grows)
                pltpu.sync_copy(grows, out_sp.at[idx_dst], add=True)

                @pl.when(b0 + 1 < nb)
                def _():
                    pltpu.make_async_copy(hs_h.at[idx_row2], grows2,
                                          sem2).wait()

                    @pl.when(b0 + 2 < nb)
                    def _():
                        prep((b0 + 2) * 128, idx_row, idx_dst)
                        pltpu.async_copy(hs_h.at[idx_row], grows, sem)
                    mul((b0 + 1) * 128, grows2)
                    pltpu.sync_copy(grows2, out_sp.at[idx_dst2], add=True)
                return c
            lax.fori_loop(0, (nb + 1) // 2, pair, 0)
            plsc.subcore_barrier()
            roff = pl.multiple_of(rb, 8)
            pltpu.sync_copy(out_sp.at[pl.ds(roff, 128)], grows)
            pltpu.sync_copy(
                grows, agg_h.at[pl.ds(pl.multiple_of(lo + roff, 8), 128)])
            roff = pl.multiple_of(rb + 128, 8)
            pltpu.sync_copy(out_sp.at[pl.ds(roff, 104)],
                            grows.at[pl.ds(0, 104)])
            pltpu.sync_copy(
                grows.at[pl.ds(0, 104)],
                agg_h.at[pl.ds(pl.multiple_of(lo + roff, 8), 104)])
            plsc.subcore_barrier()
            return c9
        lax.fori_loop(0, 7, tbody, 0)


def _dec_body(rowl_h, coll_h, zu_h, zi_h, he_h, idx_u, idx_i, gu, gi, sem):
    cid = lax.axis_index("c")
    sid = lax.axis_index("s")
    wid = sid * 2 + cid
    base = wid * (ELPAD // 32)

    def body(b, c):
        off = pl.multiple_of(base + b * 128, 128)
        pltpu.sync_copy(rowl_h.at[pl.ds(off, 128)], idx_u)
        pltpu.sync_copy(coll_h.at[pl.ds(off, 128)], idx_i)
        pltpu.async_copy(zu_h.at[idx_u], gu, sem).wait()
        pltpu.async_copy(zi_h.at[idx_i], gi, sem).wait()

        def addr(r4, c2):
            for r2 in range(4):
                r = r4 * 4 + r2
                for k in range(8):
                    sl = pl.ds(k * 16, 16)
                    gu[r, sl] = gu[r, sl] + gi[r, sl]
            return c2
        lax.fori_loop(0, 32, addr, 0)
        pltpu.sync_copy(gu, he_h.at[pl.ds(off, 128)])
        return c
    lax.fori_loop(0, ELPAD // 32 // 128, body, 0)


def _mesh():
    return plsc.VectorSubcoreMesh(core_axis_name="c", subcore_axis_name="s")


_SC_PARAMS = pltpu.CompilerParams(needs_layout_passes=False)


def _edge2_call(s0, d0, row0, col0, s1, d1, row1, col1):
    return pl.kernel(
        _edge2_body,
        out_type=[jax.ShapeDtypeStruct((EPAD,), f32),
                  jax.ShapeDtypeStruct((EPAD,), f32),
                  jax.ShapeDtypeStruct((4 * NPAD,), f32)],
        mesh=_mesh(),
        compiler_params=_SC_PARAMS,
        scratch_types=[
            pltpu.VMEM((N,), f32),
            pltpu.VMEM((N,), f32),
            pltpu.VMEM((2048,), i32),
            pltpu.VMEM((2048,), i32),
            pltpu.VMEM((16, 128), i32),
            pltpu.VMEM((2048,), f32),
            pltpu.VMEM((NPAD // 16,), f32),
            pltpu.VMEM_SHARED((NPAD,), f32),
            pltpu.VMEM_SHARED((NPAD,), f32),
        ],
    )(s0, d0, row0, col0, s1, d1, row1, col1)


def _alpha2_call(col0, eraw0, col1, eraw1, den4):
    return pl.kernel(
        _alpha2_body,
        out_type=[jax.ShapeDtypeStruct((EPAD,), f32),
                  jax.ShapeDtypeStruct((EPAD,), f32)],
        mesh=_mesh(),
        compiler_params=_SC_PARAMS,
        scratch_types=[
            pltpu.VMEM((NPAD,), f32),
            pltpu.VMEM((NPAD,), f32),
            pltpu.VMEM((2048,), i32),
            pltpu.VMEM((2048,), f32),
            pltpu.VMEM((2048,), f32),
        ],
    )(col0, eraw0, col1, eraw1, den4)


def _feat2_call(row0, col0, alpha0, hs0, row1, col1, alpha1, hs1):
    return pl.kernel(
        _feat2_body,
        out_type=[jax.ShapeDtypeStruct((FAGG, H), f32),
                  jax.ShapeDtypeStruct((FAGG, H), f32)],
        mesh=_mesh(),
        compiler_params=_SC_PARAMS,
        scratch_types=[
            pltpu.VMEM((2048,), i32),
            pltpu.VMEM((2048,), i32),
            pltpu.VMEM((2048,), f32),
            pltpu.VMEM((20608,), i32),
            pltpu.VMEM((20608,), f32),
            pltpu.VMEM((128,), i32),
            pltpu.VMEM((128,), i32),
            pltpu.VMEM((128,), i32),
            pltpu.VMEM((128,), i32),
            pltpu.VMEM((128, H), f32),
            pltpu.VMEM((128, H), f32),
            pltpu.VMEM((128, H), f32),
            pltpu.VMEM_SHARED((FCH, H), f32),
            pltpu.SemaphoreType.DMA,
            pltpu.SemaphoreType.DMA,
        ],
    )(row0, col0, alpha0, hs0, row1, col1, alpha1, hs1)


def _dec_call(rowl, coll, Zu, Zi):
    return pl.kernel(
        _dec_body,
        out_type=[jax.ShapeDtypeStruct((ELPAD, H), f32)],
        mesh=_mesh(),
        compiler_params=_SC_PARAMS,
        scratch_types=[
            pltpu.VMEM((128,), i32),
            pltpu.VMEM((128,), i32),
            pltpu.VMEM((128, H), f32),
            pltpu.VMEM((128, H), f32),
            pltpu.SemaphoreType.DMA,
        ],
    )(rowl, coll, Zu, Zi)[0]


def _run_layer(s0, d0, hs0, s1, d1, hs1,
               row0, col0, row1, col1):
    eraw0, eraw1, den4 = _edge2_call(s0, d0, row0, col0,
                                     s1, d1, row1, col1)
    alpha0, alpha1 = _alpha2_call(col0, eraw0, col1, eraw1, den4)
    agg0, agg1 = _feat2_call(row0, col0, alpha0, hs0,
                             row1, col1, alpha1, hs1)
    return agg0, agg1, alpha0, alpha1


def kernel(x_user, x_item, edge_index_ui, edge_index_iu, edge_label_index,
           params):
    p = params
    def padE(x):
        return jnp.pad(x.astype(i32), (0, EPAD - E_N), mode='wrap')

    def padL(x):
        return jnp.pad(x.astype(i32), (0, ELPAD - EL_N), mode='wrap')

    row_ui = padE(edge_index_ui[0])
    col_ui = padE(edge_index_ui[1])
    row_iu = padE(edge_index_iu[0])
    col_iu = padE(edge_index_iu[1])
    rowl = padL(edge_label_index[0])
    coll = padL(edge_label_index[1])

    c1ui, c1iu = p['conv1_ui'], p['conv1_iu']
    c2ui, c2iu = p['conv2_ui'], p['conv2_iu']
    lu, li = p['lin1_user'], p['lin1_item']

    nt = N // TB
    v = lambda x: x.reshape(H, 1)
    b = lambda x: x.reshape(1, H)
    sds = jax.ShapeDtypeStruct

    (hsu, su1, diu1, hsi, si1, dui1, linu, lini) = pl.pallas_call(
        _pre1_body,
        grid=(nt,),
        in_specs=[_rows(H), _rows(H)] + [_full((H, H)), _full((H, 1))] * 4
        + [_full((H, H)), _full((1, H))] * 2,
        out_specs=[_rows(H), _rows(1), _rows(1), _rows(H), _rows(1),
                   _rows(1), _rows(H), _rows(H)],
        out_shape=[sds((N, H), f32), sds((N, 1), f32), sds((N, 1), f32),
                   sds((N, H), f32), sds((N, 1), f32), sds((N, 1), f32),
                   sds((N, H), f32), sds((N, H), f32)],
    )(x_user, x_item,
      c1ui['Ws'], v(c1ui['as']), c1ui['Wd'], v(c1ui['ad']),
      c1iu['Ws'], v(c1iu['as']), c1iu['Wd'], v(c1iu['ad']),
      lu['W'], b(lu['b']), li['W'], b(li['b']))

    agg_i1, agg_u1, _, _ = _run_layer(
        su1.reshape(-1), dui1.reshape(-1), hsu,
        si1.reshape(-1), diu1.reshape(-1), hsi,
        row_ui, col_ui, row_iu, col_iu)

    (hs2u, s2u, d2iu, hs2i, s2i, d2ui) = pl.pallas_call(
        _mid_body,
        grid=(nt,),
        in_specs=[_rows(H)] * 4 + [_full((1, H))] * 2
        + [_full((H, H)), _full((H, 1))] * 4,
        out_specs=[_rows(H), _rows(1), _rows(1), _rows(H), _rows(1),
                   _rows(1)],
        out_shape=[sds((N, H), f32), sds((N, 1), f32), sds((N, 1), f32),
                   sds((N, H), f32), sds((N, 1), f32), sds((N, 1), f32)],
    )(agg_i1[:N], agg_u1[:N], lini, linu, b(c1ui['b']), b(c1iu['b']),
      c2ui['Ws'], v(c2ui['as']), c2ui['Wd'], v(c2ui['ad']),
      c2iu['Ws'], v(c2iu['as']), c2iu['Wd'], v(c2iu['ad']))

    agg_zi, agg_zu, alpha_ui, alpha_iu = _run_layer(
        s2u.reshape(-1), d2ui.reshape(-1), hs2u,
        s2i.reshape(-1), d2iu.reshape(-1), hs2i,
        row_ui, col_ui, row_iu, col_iu)

    Wd1 = p['dec1']['W']
    Zu, Zi = pl.pallas_call(
        _decpre_body,
        grid=(nt,),
        in_specs=[_rows(H), _rows(H), _full((1, H)), _full((1, H)),
                  _full((H, H)), _full((H, H)), _full((1, H))],
        out_specs=[_rows(H), _rows(H)],
        out_shape=[sds((N, H), f32), sds((N, H), f32)],
    )(agg_zu[:N], agg_zi[:N], b(c2iu['b']), b(c2ui['b']),
      Wd1[:H], Wd1[H:], b(p['dec1']['b']))

    He = _dec_call(rowl, coll, Zu, Zi)

    predp = pl.pallas_call(
        _decpost_body,
        grid=(ELPAD // 1024,),
        in_specs=[pl.BlockSpec((1024, H), lambda i: (i, 0)),
                  _full((H, 1)), _full((1, 1))],
        out_specs=pl.BlockSpec((1024, 1), lambda i: (i, 0)),
        out_shape=sds((ELPAD, 1), f32),
    )(He, p['dec2']['W'], p['dec2']['b'].reshape(1, 1))

    pred = predp[:EL_N, 0]
    return pred, alpha_ui[:E_N], alpha_iu[:E_N]


# speedup vs baseline: 1.2074x; 1.1225x over previous
import jax
import jax.numpy as jnp
from jax import lax
from jax.experimental import pallas as pl
from jax.experimental.pallas import tpu as pltpu
from jax.experimental.pallas import tpu_sc as plsc

H = 128
HH = 64
N = 50000
E_N = 300000
EL_N = 200000
EPAD = 327680
ELPAD = 204800
NPAD = 50176
FCH = 5120
FAGG = 51200
FTR = FCH // 16
TB = 1000

f32 = jnp.float32
i32 = jnp.int32


def _dot(a, b):
    return jnp.dot(a, b, preferred_element_type=f32)


def _pre1_body(xu, xi, Wsui, aui, Wdui, adui, Wsiu, aiu, Wdiu, adiu,
               Wlu, blu, Wli, bli,
               hsu_o, su_o, diu_o, hsi_o, si_o, dui_o,
               linu_o, lini_o):
    xu_ = xu[:]
    xi_ = xi[:]
    hsu = _dot(xu_, Wsui[:])
    hsu_o[:] = hsu
    su_o[:] = _dot(hsu, aui[:])
    diu_o[:] = _dot(_dot(xu_, Wdiu[:]), adiu[:])
    hsi = _dot(xi_, Wsiu[:])
    hsi_o[:] = hsi
    si_o[:] = _dot(hsi, aiu[:])
    dui_o[:] = _dot(_dot(xi_, Wdui[:]), adui[:])
    linu_o[:] = _dot(xu_, Wlu[:]) + blu[:]
    lini_o[:] = _dot(xi_, Wli[:]) + bli[:]


def _mid_body(aggi, aggu, lini, linu, b1ui, b1iu,
              Ws2ui, as2ui, Wd2ui, ad2ui, Ws2iu, as2iu, Wd2iu, ad2iu,
              hs2u_o, s2u_o, d2iu_o, hs2i_o, s2i_o, d2ui_o):
    hi = jnp.maximum(aggi[:] + b1ui[:] + lini[:], 0.0)
    hu = jnp.maximum(aggu[:] + b1iu[:] + linu[:], 0.0)
    hs2u = _dot(hu, Ws2ui[:])
    hs2u_o[:] = hs2u
    s2u_o[:] = _dot(hs2u, as2ui[:])
    d2iu_o[:] = _dot(_dot(hu, Wd2iu[:]), ad2iu[:])
    hs2i = _dot(hi, Ws2iu[:])
    hs2i_o[:] = hs2i
    s2i_o[:] = _dot(hs2i, as2iu[:])
    d2ui_o[:] = _dot(_dot(hi, Wd2ui[:]), ad2ui[:])


def _decpre_body(aggzu, aggzi, b2iu, b2ui, Wtop, Wbot, b1d, Zu_o, Zi_o):
    Zu_o[:] = _dot(aggzu[:] + b2iu[:], Wtop[:]) + b1d[:]
    Zi_o[:] = _dot(aggzi[:] + b2ui[:], Wbot[:])


def _decpost_body(He, w2, b2, out_o):
    h = jnp.maximum(He[:], 0.0)
    z = _dot(h, w2[:]) + b2[:]
    out_o[:] = jax.nn.sigmoid(z)


def _full(shape):
    return pl.BlockSpec(shape, lambda i: (0, 0))


def _rows(width):
    return pl.BlockSpec((TB, width), lambda i: (i, 0))


def _edge2_body(s0_h, d0_h, row0_h, col0_h,
                s1_h, d1_h, row1_h, col1_h,
                eraw0_h, eraw1_h, den_h,
                s_v, d_v, row_v, col_v, col2d_v, eraw_v, zbuf,
                den_sp0, den_sp1):
    cid = lax.axis_index("c")
    sid = lax.axis_index("s")
    wid = sid * 2 + cid

    def zb(k, c):
        zbuf[pl.ds(k * 16, 16)] = jnp.zeros((16,), f32)
        return c
    lax.fori_loop(0, NPAD // 16 // 16, zb, 0)
    dslc = pl.ds(pl.multiple_of(sid * (NPAD // 16), 8), NPAD // 16)
    pltpu.sync_copy(zbuf, den_sp0.at[dslc])
    pltpu.sync_copy(zbuf, den_sp1.at[dslc])
    plsc.subcore_barrier()

    ebase = wid * (EPAD // 32)
    for q, (s_h, d_h, row_h, col_h, eraw_h, den_sp) in enumerate([
            (s0_h, d0_h, row0_h, col0_h, eraw0_h, den_sp0),
            (s1_h, d1_h, row1_h, col1_h, eraw1_h, den_sp1)]):
        pltpu.sync_copy(s_h, s_v)
        pltpu.sync_copy(d_h, d_v)

        def chbody(ch, c0):
            cbase = pl.multiple_of(ebase + ch * 2048, 2048)
            pltpu.sync_copy(row_h.at[pl.ds(cbase, 2048)], row_v)
            pltpu.sync_copy(col_h.at[pl.ds(cbase, 2048)], col_v)

            def body(g, c):
                sl = pl.ds(g * 16, 16)
                rv = row_v[sl]
                cv = col_v[sl]
                sv = plsc.load_gather(s_v, [rv])
                dv = plsc.load_gather(d_v, [cv])
                a = sv + dv
                a = jnp.where(a > 0, a, 0.2 * a)
                e = jnp.exp(a)
                eid = cbase + g * 16 + lax.iota(i32, 16)
                e = jnp.where(eid < E_N, e, 0.0)
                eraw_v[sl] = e
                col2d_v[g // 8, pl.ds((g % 8) * 16, 16)] = cv
                return c
            lax.fori_loop(0, 128, body, 0)
            pltpu.sync_copy(eraw_v, eraw_h.at[pl.ds(cbase, 2048)])

            def kbody(k, c2):
                pltpu.sync_copy(
                    eraw_v.at[pl.ds(pl.multiple_of(k * 128, 128), 128)],
                    den_sp.at[col2d_v.at[k]], add=True)
                return c2
            lax.fori_loop(0, 16, kbody, 0)
            return c0
        lax.fori_loop(0, 5, chbody, 0)
    plsc.subcore_barrier()
    for q, den_sp in enumerate([den_sp0, den_sp1]):
        doff = pl.multiple_of((cid * 2 + q) * NPAD + sid * (NPAD // 16), 8)
        pltpu.sync_copy(den_sp.at[dslc], zbuf)
        pltpu.sync_copy(zbuf, den_h.at[pl.ds(doff, NPAD // 16)])


def _alpha2_body(col0_h, eraw0_h, col1_h, eraw1_h, den_h,
                 alpha0_h, alpha1_h,
                 den_v, den2_v, col_v, e_v, a_v):
    cid = lax.axis_index("c")
    sid = lax.axis_index("s")
    wid = sid * 2 + cid
    ebase = wid * (EPAD // 32)
    for q, (col_h, eraw_h, alpha_h) in enumerate([
            (col0_h, eraw0_h, alpha0_h), (col1_h, eraw1_h, alpha1_h)]):
        pltpu.sync_copy(den_h.at[pl.ds(q * NPAD, NPAD)], den_v)
        pltpu.sync_copy(den_h.at[pl.ds((2 + q) * NPAD, NPAD)], den2_v)

        def addb(k, c):
            sl = pl.ds(k * 16, 16)
            den_v[sl] = den_v[sl] + den2_v[sl]
            return c
        lax.fori_loop(0, NPAD // 16, addb, 0)

        def chbody(ch, c0):
            cbase = pl.multiple_of(ebase + ch * 2048, 2048)
            pltpu.sync_copy(col_h.at[pl.ds(cbase, 2048)], col_v)
            pltpu.sync_copy(eraw_h.at[pl.ds(cbase, 2048)], e_v)

            def body(g, c):
                sl = pl.ds(g * 16, 16)
                cv = col_v[sl]
                ev = e_v[sl]
                dv = plsc.load_gather(den_v, [cv])
                a_v[sl] = ev / (dv + 1e-16)
                return c
            lax.fori_loop(0, 128, body, 0)
            pltpu.sync_copy(a_v, alpha_h.at[pl.ds(cbase, 2048)])
            return c0
        lax.fori_loop(0, 5, chbody, 0)


def _feat2_body(row0_h, col0_h, alpha0_h, hs0_h,
                row1_h, col1_h, alpha1_h, hs1_h,
                agg0_h, agg1_h,
                row_v, col_v, al_v, comp_pack, comp_al,
                idx_row, idx_dst, idx_row2, idx_dst2, grows, grows2, zbuf,
                out_sp, sem, sem2):
    cid = lax.axis_index("c")
    sid = lax.axis_index("s")

    def zb(r, c):
        for k in range(8):
            zbuf[r, pl.ds(k * 16, 16)] = jnp.zeros((16,), f32)
        return c
    lax.fori_loop(0, 64, zb, 0)

    rb = pl.multiple_of(sid * FTR, 8)
    for q, (row_h, col_h, alpha_h, hs_h, agg_h) in enumerate([
            (row0_h, col0_h, alpha0_h, hs0_h, agg0_h),
            (row1_h, col1_h, alpha1_h, hs1_h, agg1_h)]):

        def tbody(t, c9):
            ck = cid * 5 + t
            lo = pl.multiple_of(ck * FCH, 128)
            hi = lo + FCH

            for i in range(5):
                pltpu.sync_copy(
                    zbuf,
                    out_sp.at[pl.ds(pl.multiple_of(rb + i * 64, 8), 64)])
            plsc.subcore_barrier()

            sbase = sid * (EPAD // 16)

            def chbody(ch, ptr):
                cbase = pl.multiple_of(sbase + ch * 2048, 2048)
                pltpu.sync_copy(row_h.at[pl.ds(cbase, 2048)], row_v)
                pltpu.sync_copy(col_h.at[pl.ds(cbase, 2048)], col_v)
                pltpu.sync_copy(alpha_h.at[pl.ds(cbase, 2048)], al_v)

                def sbody(g2, ptr):
                    for g in (g2 * 2, g2 * 2 + 1):
                        sl = pl.ds(g * 16, 16)
                        cv = col_v[sl]
                        rv = row_v[sl]
                        av = al_v[sl]
                        m = (cv >= lo) & (cv < hi)
                        mi = jnp.where(m, 1, 0).astype(i32)
                        pk = rv + ((cv - lo) << 16)
                        psl = pl.ds(ptr, 16)
                        plsc.store_compressed(comp_pack.at[psl], pk, mask=m)
                        plsc.store_compressed(comp_al.at[psl], av, mask=m)
                        ptr = ptr + jnp.sum(mi)
                    return ptr
                return lax.fori_loop(0, 64, sbody, ptr)
            ptr = lax.fori_loop(0, 10, chbody, jnp.int32(0))

            cntp = ((ptr + 127) // 128) * 128
            zi16 = jnp.zeros((16,), i32)
            zf16 = jnp.zeros((16,), f32)

            def pbody(i, c):
                idxs = ptr + i * 16 + lax.iota(i32, 16)
                pm = idxs < cntp
                plsc.store_scatter(comp_pack, [idxs], zi16, mask=pm)
                plsc.store_scatter(comp_al, [idxs], zf16, mask=pm)
                return c
            lax.fori_loop(0, 8, pbody, 0)

            nb = cntp // 128

            def prep(off, idxr, idxd):
                def cp(i, c2):
                    s16 = pl.ds(off + i * 16, 16)
                    d16 = pl.ds(i * 16, 16)
                    pk = comp_pack[s16]
                    idxr[d16] = pk & 0xFFFF
                    idxd[d16] = pk >> 16
                    return c2
                lax.fori_loop(0, 8, cp, 0)

            def mul(off, g):
                def mul_r(r4, c2):
                    for r2 in range(4):
                        r = r4 * 4 + r2
                        av = plsc.load_gather(
                            comp_al, [jnp.full((16,), off + r, i32)])
                        for k in range(8):
                            sl = pl.ds(k * 16, 16)
                            g[r, sl] = g[r, sl] * av
                    return c2
                lax.fori_loop(0, 32, mul_r, 0)

            @pl.when(nb > 0)
            def _():
                prep(0, idx_row, idx_dst)
                pltpu.async_copy(hs_h.at[idx_row], grows, sem)

            def pair(i, c):
                b0 = i * 2
                pltpu.make_async_copy(hs_h.at[idx_row], grows, sem).wait()

                @pl.when(b0 + 1 < nb)
                def _():
                    prep((b0 + 1) * 128, idx_row2, idx_dst2)
                    pltpu.async_copy(hs_h.at[idx_row2], grows2, sem2)
                mul(b0 * 128, grows)
                pltpu.sync_copy(grows, out_sp.at[idx_dst], add=True)

                @pl.when(b0 + 1 < nb)
                def _():
                    pltpu.make_async_copy(hs_h.at[idx_row2], grows2,
                                          sem2).wait()

                    @pl.when(b0 + 2 < nb)
                    def _():
                        prep((b0 + 2) * 128, idx_row, idx_dst)
                        pltpu.async_copy(hs_h.at[idx_row], grows, sem)
                    mul((b0 + 1) * 128, grows2)
                    pltpu.sync_copy(grows2, out_sp.at[idx_dst2], add=True)
                return c
            lax.fori_loop(0, (nb + 1) // 2, pair, 0)
            plsc.subcore_barrier()
            for i in range(2):
                roff = pl.multiple_of(rb + i * 128, 8)
                pltpu.sync_copy(out_sp.at[pl.ds(roff, 128)], grows)
                pltpu.sync_copy(
                    grows, agg_h.at[pl.ds(pl.multiple_of(lo + roff, 8),
                                          128)])
            roff = pl.multiple_of(rb + 256, 8)
            pltpu.sync_copy(out_sp.at[pl.ds(roff, 64)],
                            grows.at[pl.ds(0, 64)])
            pltpu.sync_copy(
                grows.at[pl.ds(0, 64)],
                agg_h.at[pl.ds(pl.multiple_of(lo + roff, 8), 64)])
            plsc.subcore_barrier()
            return c9
        lax.fori_loop(0, 5, tbody, 0)


def _dec_body(rowl_h, coll_h, zu_h, zi_h, he_h, idx_u, idx_i, gu, gi, sem):
    cid = lax.axis_index("c")
    sid = lax.axis_index("s")
    wid = sid * 2 + cid
    base = wid * (ELPAD // 32)

    def body(b, c):
        off = pl.multiple_of(base + b * 128, 128)
        pltpu.sync_copy(rowl_h.at[pl.ds(off, 128)], idx_u)
        pltpu.sync_copy(coll_h.at[pl.ds(off, 128)], idx_i)
        pltpu.async_copy(zu_h.at[idx_u], gu, sem).wait()
        pltpu.async_copy(zi_h.at[idx_i], gi, sem).wait()

        def addr(r4, c2):
            for r2 in range(4):
                r = r4 * 4 + r2
                for k in range(8):
                    sl = pl.ds(k * 16, 16)
                    gu[r, sl] = gu[r, sl] + gi[r, sl]
            return c2
        lax.fori_loop(0, 32, addr, 0)
        pltpu.sync_copy(gu, he_h.at[pl.ds(off, 128)])
        return c
    lax.fori_loop(0, ELPAD // 32 // 128, body, 0)


def _mesh():
    return plsc.VectorSubcoreMesh(core_axis_name="c", subcore_axis_name="s")


_SC_PARAMS = pltpu.CompilerParams(needs_layout_passes=False)


def _edge2_call(s0, d0, row0, col0, s1, d1, row1, col1):
    return pl.kernel(
        _edge2_body,
        out_type=[jax.ShapeDtypeStruct((EPAD,), f32),
                  jax.ShapeDtypeStruct((EPAD,), f32),
                  jax.ShapeDtypeStruct((4 * NPAD,), f32)],
        mesh=_mesh(),
        compiler_params=_SC_PARAMS,
        scratch_types=[
            pltpu.VMEM((N,), f32),
            pltpu.VMEM((N,), f32),
            pltpu.VMEM((2048,), i32),
            pltpu.VMEM((2048,), i32),
            pltpu.VMEM((16, 128), i32),
            pltpu.VMEM((2048,), f32),
            pltpu.VMEM((NPAD // 16,), f32),
            pltpu.VMEM_SHARED((NPAD,), f32),
            pltpu.VMEM_SHARED((NPAD,), f32),
        ],
    )(s0, d0, row0, col0, s1, d1, row1, col1)


def _alpha2_call(col0, eraw0, col1, eraw1, den4):
    return pl.kernel(
        _alpha2_body,
        out_type=[jax.ShapeDtypeStruct((EPAD,), f32),
                  jax.ShapeDtypeStruct((EPAD,), f32)],
        mesh=_mesh(),
        compiler_params=_SC_PARAMS,
        scratch_types=[
            pltpu.VMEM((NPAD,), f32),
            pltpu.VMEM((NPAD,), f32),
            pltpu.VMEM((2048,), i32),
            pltpu.VMEM((2048,), f32),
            pltpu.VMEM((2048,), f32),
        ],
    )(col0, eraw0, col1, eraw1, den4)


def _feat2_call(row0, col0, alpha0, hs0, row1, col1, alpha1, hs1):
    return pl.kernel(
        _feat2_body,
        out_type=[jax.ShapeDtypeStruct((FAGG, H), f32),
                  jax.ShapeDtypeStruct((FAGG, H), f32)],
        mesh=_mesh(),
        compiler_params=_SC_PARAMS,
        scratch_types=[
            pltpu.VMEM((2048,), i32),
            pltpu.VMEM((2048,), i32),
            pltpu.VMEM((2048,), f32),
            pltpu.VMEM((20608,), i32),
            pltpu.VMEM((20608,), f32),
            pltpu.VMEM((128,), i32),
            pltpu.VMEM((128,), i32),
            pltpu.VMEM((128,), i32),
            pltpu.VMEM((128,), i32),
            pltpu.VMEM((128, H), f32),
            pltpu.VMEM((128, H), f32),
            pltpu.VMEM((64, H), f32),
            pltpu.VMEM_SHARED((FCH, H), f32),
            pltpu.SemaphoreType.DMA,
            pltpu.SemaphoreType.DMA,
        ],
    )(row0, col0, alpha0, hs0, row1, col1, alpha1, hs1)


def _dec_call(rowl, coll, Zu, Zi):
    return pl.kernel(
        _dec_body,
        out_type=[jax.ShapeDtypeStruct((ELPAD, H), f32)],
        mesh=_mesh(),
        compiler_params=_SC_PARAMS,
        scratch_types=[
            pltpu.VMEM((128,), i32),
            pltpu.VMEM((128,), i32),
            pltpu.VMEM((128, H), f32),
            pltpu.VMEM((128, H), f32),
            pltpu.SemaphoreType.DMA,
        ],
    )(rowl, coll, Zu, Zi)[0]


def _run_layer(s0, d0, hs0, s1, d1, hs1,
               row0, col0, row1, col1):
    eraw0, eraw1, den4 = _edge2_call(s0, d0, row0, col0,
                                     s1, d1, row1, col1)
    alpha0, alpha1 = _alpha2_call(col0, eraw0, col1, eraw1, den4)
    agg0, agg1 = _feat2_call(row0, col0, alpha0, hs0,
                             row1, col1, alpha1, hs1)
    return agg0, agg1, alpha0, alpha1


def kernel(x_user, x_item, edge_index_ui, edge_index_iu, edge_label_index,
           params):
    p = params
    def padE(x):
        return jnp.pad(x.astype(i32), (0, EPAD - E_N), mode='wrap')

    def padL(x):
        return jnp.pad(x.astype(i32), (0, ELPAD - EL_N), mode='wrap')

    row_ui = padE(edge_index_ui[0])
    col_ui = padE(edge_index_ui[1])
    row_iu = padE(edge_index_iu[0])
    col_iu = padE(edge_index_iu[1])
    rowl = padL(edge_label_index[0])
    coll = padL(edge_label_index[1])

    c1ui, c1iu = p['conv1_ui'], p['conv1_iu']
    c2ui, c2iu = p['conv2_ui'], p['conv2_iu']
    lu, li = p['lin1_user'], p['lin1_item']

    nt = N // TB
    v = lambda x: x.reshape(H, 1)
    b = lambda x: x.reshape(1, H)
    sds = jax.ShapeDtypeStruct

    (hsu, su1, diu1, hsi, si1, dui1, linu, lini) = pl.pallas_call(
        _pre1_body,
        grid=(nt,),
        in_specs=[_rows(H), _rows(H)] + [_full((H, H)), _full((H, 1))] * 4
        + [_full((H, H)), _full((1, H))] * 2,
        out_specs=[_rows(H), _rows(1), _rows(1), _rows(H), _rows(1),
                   _rows(1), _rows(H), _rows(H)],
        out_shape=[sds((N, H), f32), sds((N, 1), f32), sds((N, 1), f32),
                   sds((N, H), f32), sds((N, 1), f32), sds((N, 1), f32),
                   sds((N, H), f32), sds((N, H), f32)],
    )(x_user, x_item,
      c1ui['Ws'], v(c1ui['as']), c1ui['Wd'], v(c1ui['ad']),
      c1iu['Ws'], v(c1iu['as']), c1iu['Wd'], v(c1iu['ad']),
      lu['W'], b(lu['b']), li['W'], b(li['b']))

    agg_i1, agg_u1, _, _ = _run_layer(
        su1.reshape(-1), dui1.reshape(-1), hsu,
        si1.reshape(-1), diu1.reshape(-1), hsi,
        row_ui, col_ui, row_iu, col_iu)

    (hs2u, s2u, d2iu, hs2i, s2i, d2ui) = pl.pallas_call(
        _mid_body,
        grid=(nt,),
        in_specs=[_rows(H)] * 4 + [_full((1, H))] * 2
        + [_full((H, H)), _full((H, 1))] * 4,
        out_specs=[_rows(H), _rows(1), _rows(1), _rows(H), _rows(1),
                   _rows(1)],
        out_shape=[sds((N, H), f32), sds((N, 1), f32), sds((N, 1), f32),
                   sds((N, H), f32), sds((N, 1), f32), sds((N, 1), f32)],
    )(agg_i1[:N], agg_u1[:N], lini, linu, b(c1ui['b']), b(c1iu['b']),
      c2ui['Ws'], v(c2ui['as']), c2ui['Wd'], v(c2ui['ad']),
      c2iu['Ws'], v(c2iu['as']), c2iu['Wd'], v(c2iu['ad']))

    agg_zi, agg_zu, alpha_ui, alpha_iu = _run_layer(
        s2u.reshape(-1), d2ui.reshape(-1), hs2u,
        s2i.reshape(-1), d2iu.reshape(-1), hs2i,
        row_ui, col_ui, row_iu, col_iu)

    Wd1 = p['dec1']['W']
    Zu, Zi = pl.pallas_call(
        _decpre_body,
        grid=(nt,),
        in_specs=[_rows(H), _rows(H), _full((1, H)), _full((1, H)),
                  _full((H, H)), _full((H, H)), _full((1, H))],
        out_specs=[_rows(H), _rows(H)],
        out_shape=[sds((N, H), f32), sds((N, H), f32)],
    )(agg_zu[:N], agg_zi[:N], b(c2iu['b']), b(c2ui['b']),
      Wd1[:H], Wd1[H:], b(p['dec1']['b']))

    He = _dec_call(rowl, coll, Zu, Zi)

    predp = pl.pallas_call(
        _decpost_body,
        grid=(ELPAD // 1024,),
        in_specs=[pl.BlockSpec((1024, H), lambda i: (i, 0)),
                  _full((H, 1)), _full((1, 1))],
        out_specs=pl.BlockSpec((1024, 1), lambda i: (i, 0)),
        out_shape=sds((ELPAD, 1), f32),
    )(He, p['dec2']['W'], p['dec2']['b'].reshape(1, 1))

    pred = predp[:EL_N, 0]
    return pred, alpha_ui[:E_N], alpha_iu[:E_N]


# speedup vs baseline: 1.2481x; 1.0337x over previous
import jax
import jax.numpy as jnp
from jax import lax
from jax.experimental import pallas as pl
from jax.experimental.pallas import tpu as pltpu
from jax.experimental.pallas import tpu_sc as plsc

H = 128
HH = 64
N = 50000
E_N = 300000
EL_N = 200000
EPAD = 327680
ELPAD = 204800
NPAD = 50176
FCH = 5120
FAGG = 51200
FTR = FCH // 16
TB = 1000

f32 = jnp.float32
i32 = jnp.int32


def _dot(a, b):
    return jnp.dot(a, b, preferred_element_type=f32)


def _pre1_body(xu, xi, Wsui, aui, Wdui, adui, Wsiu, aiu, Wdiu, adiu,
               Wlu, blu, Wli, bli,
               hsu_o, su_o, diu_o, hsi_o, si_o, dui_o,
               linu_o, lini_o):
    xu_ = xu[:]
    xi_ = xi[:]
    hsu = _dot(xu_, Wsui[:])
    hsu_o[:] = hsu
    su_o[:] = _dot(hsu, aui[:])
    diu_o[:] = _dot(_dot(xu_, Wdiu[:]), adiu[:])
    hsi = _dot(xi_, Wsiu[:])
    hsi_o[:] = hsi
    si_o[:] = _dot(hsi, aiu[:])
    dui_o[:] = _dot(_dot(xi_, Wdui[:]), adui[:])
    linu_o[:] = _dot(xu_, Wlu[:]) + blu[:]
    lini_o[:] = _dot(xi_, Wli[:]) + bli[:]


def _mid_body(aggi, aggu, lini, linu, b1ui, b1iu,
              Ws2ui, as2ui, Wd2ui, ad2ui, Ws2iu, as2iu, Wd2iu, ad2iu,
              hs2u_o, s2u_o, d2iu_o, hs2i_o, s2i_o, d2ui_o):
    hi = jnp.maximum(aggi[:] + b1ui[:] + lini[:], 0.0)
    hu = jnp.maximum(aggu[:] + b1iu[:] + linu[:], 0.0)
    hs2u = _dot(hu, Ws2ui[:])
    hs2u_o[:] = hs2u
    s2u_o[:] = _dot(hs2u, as2ui[:])
    d2iu_o[:] = _dot(_dot(hu, Wd2iu[:]), ad2iu[:])
    hs2i = _dot(hi, Ws2iu[:])
    hs2i_o[:] = hs2i
    s2i_o[:] = _dot(hs2i, as2iu[:])
    d2ui_o[:] = _dot(_dot(hi, Wd2ui[:]), ad2ui[:])


def _decpre_body(aggzu, aggzi, b2iu, b2ui, Wtop, Wbot, b1d, Zu_o, Zi_o):
    Zu_o[:] = _dot(aggzu[:] + b2iu[:], Wtop[:]) + b1d[:]
    Zi_o[:] = _dot(aggzi[:] + b2ui[:], Wbot[:])


def _decpost_body(He, w2, b2, out_o):
    h = jnp.maximum(He[:], 0.0)
    z = _dot(h, w2[:]) + b2[:]
    out_o[:] = jax.nn.sigmoid(z)


def _full(shape):
    return pl.BlockSpec(shape, lambda i: (0, 0))


def _rows(width):
    return pl.BlockSpec((TB, width), lambda i: (i, 0))


def _edge2_body(s0_h, d0_h, row0_h, col0_h,
                s1_h, d1_h, row1_h, col1_h,
                eraw0_h, eraw1_h, den_h,
                s_v, d_v, row_v, col_v, col2d_v, eraw_v, zbuf,
                den_sp0, den_sp1):
    cid = lax.axis_index("c")
    sid = lax.axis_index("s")
    wid = sid * 2 + cid

    def zb(k, c):
        zbuf[pl.ds(k * 16, 16)] = jnp.zeros((16,), f32)
        return c
    lax.fori_loop(0, NPAD // 16 // 16, zb, 0)
    dslc = pl.ds(pl.multiple_of(sid * (NPAD // 16), 8), NPAD // 16)
    pltpu.sync_copy(zbuf, den_sp0.at[dslc])
    pltpu.sync_copy(zbuf, den_sp1.at[dslc])
    plsc.subcore_barrier()

    ebase = wid * (EPAD // 32)
    for q, (s_h, d_h, row_h, col_h, eraw_h, den_sp) in enumerate([
            (s0_h, d0_h, row0_h, col0_h, eraw0_h, den_sp0),
            (s1_h, d1_h, row1_h, col1_h, eraw1_h, den_sp1)]):
        pltpu.sync_copy(s_h, s_v)
        pltpu.sync_copy(d_h, d_v)

        def chbody(ch, c0):
            cbase = pl.multiple_of(ebase + ch * 2048, 2048)
            pltpu.sync_copy(row_h.at[pl.ds(cbase, 2048)], row_v)
            pltpu.sync_copy(col_h.at[pl.ds(cbase, 2048)], col_v)

            def body(g, c):
                sl = pl.ds(g * 16, 16)
                rv = row_v[sl]
                cv = col_v[sl]
                sv = plsc.load_gather(s_v, [rv])
                dv = plsc.load_gather(d_v, [cv])
                a = sv + dv
                a = jnp.where(a > 0, a, 0.2 * a)
                e = jnp.exp(a)
                eid = cbase + g * 16 + lax.iota(i32, 16)
                e = jnp.where(eid < E_N, e, 0.0)
                eraw_v[sl] = e
                col2d_v[g // 8, pl.ds((g % 8) * 16, 16)] = cv
                return c
            lax.fori_loop(0, 128, body, 0)
            pltpu.sync_copy(eraw_v, eraw_h.at[pl.ds(cbase, 2048)])

            def kbody(k, c2):
                pltpu.sync_copy(
                    eraw_v.at[pl.ds(pl.multiple_of(k * 128, 128), 128)],
                    den_sp.at[col2d_v.at[k]], add=True)
                return c2
            lax.fori_loop(0, 16, kbody, 0)
            return c0
        lax.fori_loop(0, 5, chbody, 0)
    plsc.subcore_barrier()
    for q, den_sp in enumerate([den_sp0, den_sp1]):
        doff = pl.multiple_of((cid * 2 + q) * NPAD + sid * (NPAD // 16), 8)
        pltpu.sync_copy(den_sp.at[dslc], zbuf)
        pltpu.sync_copy(zbuf, den_h.at[pl.ds(doff, NPAD // 16)])


def _alpha2_body(col0_h, eraw0_h, col1_h, eraw1_h, den_h,
                 alpha0_h, alpha1_h,
                 den_v, den2_v, col_v, e_v, a_v):
    cid = lax.axis_index("c")
    sid = lax.axis_index("s")
    wid = sid * 2 + cid
    ebase = wid * (EPAD // 32)
    for q, (col_h, eraw_h, alpha_h) in enumerate([
            (col0_h, eraw0_h, alpha0_h), (col1_h, eraw1_h, alpha1_h)]):
        pltpu.sync_copy(den_h.at[pl.ds(q * NPAD, NPAD)], den_v)
        pltpu.sync_copy(den_h.at[pl.ds((2 + q) * NPAD, NPAD)], den2_v)

        def addb(k, c):
            sl = pl.ds(k * 16, 16)
            den_v[sl] = den_v[sl] + den2_v[sl]
            return c
        lax.fori_loop(0, NPAD // 16, addb, 0)

        def chbody(ch, c0):
            cbase = pl.multiple_of(ebase + ch * 2048, 2048)
            pltpu.sync_copy(col_h.at[pl.ds(cbase, 2048)], col_v)
            pltpu.sync_copy(eraw_h.at[pl.ds(cbase, 2048)], e_v)

            def body(g, c):
                sl = pl.ds(g * 16, 16)
                cv = col_v[sl]
                ev = e_v[sl]
                dv = plsc.load_gather(den_v, [cv])
                a_v[sl] = ev / (dv + 1e-16)
                return c
            lax.fori_loop(0, 128, body, 0)
            pltpu.sync_copy(a_v, alpha_h.at[pl.ds(cbase, 2048)])
            return c0
        lax.fori_loop(0, 5, chbody, 0)


def _feat2_body(row0_h, col0_h, alpha0_h, hs0_h,
                row1_h, col1_h, alpha1_h, hs1_h,
                agg0_h, agg1_h,
                row_v, col_v, al_v, comp_pack, comp_al,
                idx_row, idx_dst, idx_row2, idx_dst2, grows, grows2, zbuf,
                out_sp, sem, sem2):
    cid = lax.axis_index("c")
    sid = lax.axis_index("s")

    def zb(r, c):
        for k in range(8):
            zbuf[r, pl.ds(k * 16, 16)] = jnp.zeros((16,), f32)
        return c
    lax.fori_loop(0, 64, zb, 0)

    rb = pl.multiple_of(sid * FTR, 8)
    for q, (row_h, col_h, alpha_h, hs_h, agg_h) in enumerate([
            (row0_h, col0_h, alpha0_h, hs0_h, agg0_h),
            (row1_h, col1_h, alpha1_h, hs1_h, agg1_h)]):

        def tbody(t, c9):
            ck = cid * 5 + t
            lo = pl.multiple_of(ck * FCH, 128)
            hi = lo + FCH

            for i in range(5):
                pltpu.sync_copy(
                    zbuf,
                    out_sp.at[pl.ds(pl.multiple_of(rb + i * 64, 8), 64)])
            plsc.subcore_barrier()

            sbase = sid * (EPAD // 16)

            def chbody(ch, ptr):
                cbase = pl.multiple_of(sbase + ch * 2048, 2048)
                pltpu.sync_copy(row_h.at[pl.ds(cbase, 2048)], row_v)
                pltpu.sync_copy(col_h.at[pl.ds(cbase, 2048)], col_v)
                pltpu.sync_copy(alpha_h.at[pl.ds(cbase, 2048)], al_v)

                def sbody(g2, ptr):
                    for g in (g2 * 2, g2 * 2 + 1):
                        sl = pl.ds(g * 16, 16)
                        cv = col_v[sl]
                        rv = row_v[sl]
                        av = al_v[sl]
                        m = (cv >= lo) & (cv < hi)
                        mi = jnp.where(m, 1, 0).astype(i32)
                        pk = rv + ((cv - lo) << 16)
                        psl = pl.ds(ptr, 16)
                        plsc.store_compressed(comp_pack.at[psl], pk, mask=m)
                        plsc.store_compressed(comp_al.at[psl], av, mask=m)
                        ptr = ptr + jnp.sum(mi)
                    return ptr
                return lax.fori_loop(0, 64, sbody, ptr)
            ptr = lax.fori_loop(0, 10, chbody, jnp.int32(0))

            cntp = ((ptr + 127) // 128) * 128
            zi16 = jnp.zeros((16,), i32)
            zf16 = jnp.zeros((16,), f32)

            def pbody(i, c):
                idxs = ptr + i * 16 + lax.iota(i32, 16)
                pm = idxs < cntp
                plsc.store_scatter(comp_pack, [idxs], zi16, mask=pm)
                plsc.store_scatter(comp_al, [idxs], zf16, mask=pm)
                return c
            lax.fori_loop(0, 8, pbody, 0)

            nb = cntp // 128

            def prep(off, idxr, idxd):
                def cp(i, c2):
                    s16 = pl.ds(off + i * 16, 16)
                    d16 = pl.ds(i * 16, 16)
                    pk = comp_pack[s16]
                    idxr[d16] = pk & 0xFFFF
                    idxd[d16] = pk >> 16
                    return c2
                lax.fori_loop(0, 8, cp, 0)

            def mul(off, g):
                def mul_r(r4, c2):
                    for r2 in range(4):
                        r = r4 * 4 + r2
                        av = plsc.load_gather(
                            comp_al, [jnp.full((16,), off + r, i32)])
                        for k in range(8):
                            sl = pl.ds(k * 16, 16)
                            g[r, sl] = g[r, sl] * av
                    return c2
                lax.fori_loop(0, 32, mul_r, 0)

            @pl.when(nb > 0)
            def _():
                prep(0, idx_row, idx_dst)
                pltpu.async_copy(hs_h.at[idx_row], grows, sem)

            def pair(i, c):
                b0 = i * 2
                pltpu.make_async_copy(hs_h.at[idx_row], grows, sem).wait()

                @pl.when(b0 + 1 < nb)
                def _():
                    prep((b0 + 1) * 128, idx_row2, idx_dst2)
                    pltpu.async_copy(hs_h.at[idx_row2], grows2, sem2)
                mul(b0 * 128, grows)
                pltpu.sync_copy(grows, out_sp.at[idx_dst], add=True)

                @pl.when(b0 + 1 < nb)
                def _():
                    pltpu.make_async_copy(hs_h.at[idx_row2], grows2,
                                          sem2).wait()

                    @pl.when(b0 + 2 < nb)
                    def _():
                        prep((b0 + 2) * 128, idx_row, idx_dst)
                        pltpu.async_copy(hs_h.at[idx_row], grows, sem)
                    mul((b0 + 1) * 128, grows2)
                    pltpu.sync_copy(grows2, out_sp.at[idx_dst2], add=True)
                return c
            lax.fori_loop(0, (nb + 1) // 2, pair, 0)
            plsc.subcore_barrier()
            for i in range(2):
                roff = pl.multiple_of(rb + i * 128, 8)
                pltpu.sync_copy(out_sp.at[pl.ds(roff, 128)], grows)
                pltpu.sync_copy(
                    grows, agg_h.at[pl.ds(pl.multiple_of(lo + roff, 8),
                                          128)])
            roff = pl.multiple_of(rb + 256, 8)
            pltpu.sync_copy(out_sp.at[pl.ds(roff, 64)],
                            grows.at[pl.ds(0, 64)])
            pltpu.sync_copy(
                grows.at[pl.ds(0, 64)],
                agg_h.at[pl.ds(pl.multiple_of(lo + roff, 8), 64)])
            plsc.subcore_barrier()
            return c9
        lax.fori_loop(0, 5, tbody, 0)


def _dec_body(rowl_h, coll_h, zu_h, zi_h, he_h,
              idx_u, idx_i, idx_u2, idx_i2, gu, gi, gu2, gi2, sem, sem2):
    cid = lax.axis_index("c")
    sid = lax.axis_index("s")
    wid = sid * 2 + cid
    base = wid * (ELPAD // 32)
    nb = ELPAD // 32 // 128

    def start(b, iu, ii, bu, bi, s):
        off = pl.multiple_of(base + b * 128, 128)
        pltpu.sync_copy(rowl_h.at[pl.ds(off, 128)], iu)
        pltpu.sync_copy(coll_h.at[pl.ds(off, 128)], ii)
        pltpu.async_copy(zu_h.at[iu], bu, s)
        pltpu.async_copy(zi_h.at[ii], bi, s)

    def finish(b, iu, ii, bu, bi, s):
        off = pl.multiple_of(base + b * 128, 128)
        pltpu.make_async_copy(zu_h.at[iu], bu, s).wait()
        pltpu.make_async_copy(zi_h.at[ii], bi, s).wait()

        def addr(r4, c2):
            for r2 in range(4):
                r = r4 * 4 + r2
                for k in range(8):
                    sl = pl.ds(k * 16, 16)
                    bu[r, sl] = bu[r, sl] + bi[r, sl]
            return c2
        lax.fori_loop(0, 32, addr, 0)
        pltpu.sync_copy(bu, he_h.at[pl.ds(off, 128)])

    start(0, idx_u, idx_i, gu, gi, sem)

    def body(p, c):
        b0 = p * 2
        start(b0 + 1, idx_u2, idx_i2, gu2, gi2, sem2)
        finish(b0, idx_u, idx_i, gu, gi, sem)

        @pl.when(b0 + 2 < nb)
        def _():
            start(b0 + 2, idx_u, idx_i, gu, gi, sem)
        finish(b0 + 1, idx_u2, idx_i2, gu2, gi2, sem2)
        return c
    lax.fori_loop(0, nb // 2, body, 0)


def _mesh():
    return plsc.VectorSubcoreMesh(core_axis_name="c", subcore_axis_name="s")


_SC_PARAMS = pltpu.CompilerParams(needs_layout_passes=False)


def _edge2_call(s0, d0, row0, col0, s1, d1, row1, col1):
    return pl.kernel(
        _edge2_body,
        out_type=[jax.ShapeDtypeStruct((EPAD,), f32),
                  jax.ShapeDtypeStruct((EPAD,), f32),
                  jax.ShapeDtypeStruct((4 * NPAD,), f32)],
        mesh=_mesh(),
        compiler_params=_SC_PARAMS,
        scratch_types=[
            pltpu.VMEM((N,), f32),
            pltpu.VMEM((N,), f32),
            pltpu.VMEM((2048,), i32),
            pltpu.VMEM((2048,), i32),
            pltpu.VMEM((16, 128), i32),
            pltpu.VMEM((2048,), f32),
            pltpu.VMEM((NPAD // 16,), f32),
            pltpu.VMEM_SHARED((NPAD,), f32),
            pltpu.VMEM_SHARED((NPAD,), f32),
        ],
    )(s0, d0, row0, col0, s1, d1, row1, col1)


def _alpha2_call(col0, eraw0, col1, eraw1, den4):
    return pl.kernel(
        _alpha2_body,
        out_type=[jax.ShapeDtypeStruct((EPAD,), f32),
                  jax.ShapeDtypeStruct((EPAD,), f32)],
        mesh=_mesh(),
        compiler_params=_SC_PARAMS,
        scratch_types=[
            pltpu.VMEM((NPAD,), f32),
            pltpu.VMEM((NPAD,), f32),
            pltpu.VMEM((2048,), i32),
            pltpu.VMEM((2048,), f32),
            pltpu.VMEM((2048,), f32),
        ],
    )(col0, eraw0, col1, eraw1, den4)


def _feat2_call(row0, col0, alpha0, hs0, row1, col1, alpha1, hs1):
    return pl.kernel(
        _feat2_body,
        out_type=[jax.ShapeDtypeStruct((FAGG, H), f32),
                  jax.ShapeDtypeStruct((FAGG, H), f32)],
        mesh=_mesh(),
        compiler_params=_SC_PARAMS,
        scratch_types=[
            pltpu.VMEM((2048,), i32),
            pltpu.VMEM((2048,), i32),
            pltpu.VMEM((2048,), f32),
            pltpu.VMEM((20608,), i32),
            pltpu.VMEM((20608,), f32),
            pltpu.VMEM((128,), i32),
            pltpu.VMEM((128,), i32),
            pltpu.VMEM((128,), i32),
            pltpu.VMEM((128,), i32),
            pltpu.VMEM((128, H), f32),
            pltpu.VMEM((128, H), f32),
            pltpu.VMEM((64, H), f32),
            pltpu.VMEM_SHARED((FCH, H), f32),
            pltpu.SemaphoreType.DMA,
            pltpu.SemaphoreType.DMA,
        ],
    )(row0, col0, alpha0, hs0, row1, col1, alpha1, hs1)


def _dec_call(rowl, coll, Zu, Zi):
    return pl.kernel(
        _dec_body,
        out_type=[jax.ShapeDtypeStruct((ELPAD, H), f32)],
        mesh=_mesh(),
        compiler_params=_SC_PARAMS,
        scratch_types=[
            pltpu.VMEM((128,), i32),
            pltpu.VMEM((128,), i32),
            pltpu.VMEM((128,), i32),
            pltpu.VMEM((128,), i32),
            pltpu.VMEM((128, H), f32),
            pltpu.VMEM((128, H), f32),
            pltpu.VMEM((128, H), f32),
            pltpu.VMEM((128, H), f32),
            pltpu.SemaphoreType.DMA,
            pltpu.SemaphoreType.DMA,
        ],
    )(rowl, coll, Zu, Zi)[0]


def _run_layer(s0, d0, hs0, s1, d1, hs1,
               row0, col0, row1, col1):
    eraw0, eraw1, den4 = _edge2_call(s0, d0, row0, col0,
                                     s1, d1, row1, col1)
    alpha0, alpha1 = _alpha2_call(col0, eraw0, col1, eraw1, den4)
    agg0, agg1 = _feat2_call(row0, col0, alpha0, hs0,
                             row1, col1, alpha1, hs1)
    return agg0, agg1, alpha0, alpha1


def kernel(x_user, x_item, edge_index_ui, edge_index_iu, edge_label_index,
           params):
    p = params
    def padE(x):
        return jnp.pad(x.astype(i32), (0, EPAD - E_N), mode='wrap')

    def padL(x):
        return jnp.pad(x.astype(i32), (0, ELPAD - EL_N), mode='wrap')

    row_ui = padE(edge_index_ui[0])
    col_ui = padE(edge_index_ui[1])
    row_iu = padE(edge_index_iu[0])
    col_iu = padE(edge_index_iu[1])
    rowl = padL(edge_label_index[0])
    coll = padL(edge_label_index[1])

    c1ui, c1iu = p['conv1_ui'], p['conv1_iu']
    c2ui, c2iu = p['conv2_ui'], p['conv2_iu']
    lu, li = p['lin1_user'], p['lin1_item']

    nt = N // TB
    v = lambda x: x.reshape(H, 1)
    b = lambda x: x.reshape(1, H)
    sds = jax.ShapeDtypeStruct

    (hsu, su1, diu1, hsi, si1, dui1, linu, lini) = pl.pallas_call(
        _pre1_body,
        grid=(nt,),
        in_specs=[_rows(H), _rows(H)] + [_full((H, H)), _full((H, 1))] * 4
        + [_full((H, H)), _full((1, H))] * 2,
        out_specs=[_rows(H), _rows(1), _rows(1), _rows(H), _rows(1),
                   _rows(1), _rows(H), _rows(H)],
        out_shape=[sds((N, H), f32), sds((N, 1), f32), sds((N, 1), f32),
                   sds((N, H), f32), sds((N, 1), f32), sds((N, 1), f32),
                   sds((N, H), f32), sds((N, H), f32)],
    )(x_user, x_item,
      c1ui['Ws'], v(c1ui['as']), c1ui['Wd'], v(c1ui['ad']),
      c1iu['Ws'], v(c1iu['as']), c1iu['Wd'], v(c1iu['ad']),
      lu['W'], b(lu['b']), li['W'], b(li['b']))

    agg_i1, agg_u1, _, _ = _run_layer(
        su1.reshape(-1), dui1.reshape(-1), hsu,
        si1.reshape(-1), diu1.reshape(-1), hsi,
        row_ui, col_ui, row_iu, col_iu)

    (hs2u, s2u, d2iu, hs2i, s2i, d2ui) = pl.pallas_call(
        _mid_body,
        grid=(nt,),
        in_specs=[_rows(H)] * 4 + [_full((1, H))] * 2
        + [_full((H, H)), _full((H, 1))] * 4,
        out_specs=[_rows(H), _rows(1), _rows(1), _rows(H), _rows(1),
                   _rows(1)],
        out_shape=[sds((N, H), f32), sds((N, 1), f32), sds((N, 1), f32),
                   sds((N, H), f32), sds((N, 1), f32), sds((N, 1), f32)],
    )(agg_i1[:N], agg_u1[:N], lini, linu, b(c1ui['b']), b(c1iu['b']),
      c2ui['Ws'], v(c2ui['as']), c2ui['Wd'], v(c2ui['ad']),
      c2iu['Ws'], v(c2iu['as']), c2iu['Wd'], v(c2iu['ad']))

    agg_zi, agg_zu, alpha_ui, alpha_iu = _run_layer(
        s2u.reshape(-1), d2ui.reshape(-1), hs2u,
        s2i.reshape(-1), d2iu.reshape(-1), hs2i,
        row_ui, col_ui, row_iu, col_iu)

    Wd1 = p['dec1']['W']
    Zu, Zi = pl.pallas_call(
        _decpre_body,
        grid=(nt,),
        in_specs=[_rows(H), _rows(H), _full((1, H)), _full((1, H)),
                  _full((H, H)), _full((H, H)), _full((1, H))],
        out_specs=[_rows(H), _rows(H)],
        out_shape=[sds((N, H), f32), sds((N, H), f32)],
    )(agg_zu[:N], agg_zi[:N], b(c2iu['b']), b(c2ui['b']),
      Wd1[:H], Wd1[H:], b(p['dec1']['b']))

    He = _dec_call(rowl, coll, Zu, Zi)

    predp = pl.pallas_call(
        _decpost_body,
        grid=(ELPAD // 1024,),
        in_specs=[pl.BlockSpec((1024, H), lambda i: (i, 0)),
                  _full((H, 1)), _full((1, 1))],
        out_specs=pl.BlockSpec((1024, 1), lambda i: (i, 0)),
        out_shape=sds((ELPAD, 1), f32),
    )(He, p['dec2']['W'], p['dec2']['b'].reshape(1, 1))

    pred = predp[:EL_N, 0]
    return pred, alpha_ui[:E_N], alpha_iu[:E_N]


# speedup vs baseline: 1.3023x; 1.0435x over previous
import jax
import jax.numpy as jnp
from jax import lax
from jax.experimental import pallas as pl
from jax.experimental.pallas import tpu as pltpu
from jax.experimental.pallas import tpu_sc as plsc

H = 128
HH = 64
N = 50000
E_N = 300000
EL_N = 200000
EPAD = 327680
ELPAD = 204800
NPAD = 50176
FCH = 5120
FAGG = 51200
FTR = FCH // 16
TB = 1000

f32 = jnp.float32
i32 = jnp.int32


def _dot(a, b):
    return jnp.dot(a, b, preferred_element_type=f32)


def _pre1_body(xu, xi, Wsui, aui, Wdui, adui, Wsiu, aiu, Wdiu, adiu,
               Wlu, blu, Wli, bli,
               hsu_o, su_o, diu_o, hsi_o, si_o, dui_o,
               linu_o, lini_o):
    xu_ = xu[:]
    xi_ = xi[:]
    hsu = _dot(xu_, Wsui[:])
    hsu_o[:] = hsu
    su_o[:] = _dot(hsu, aui[:])
    diu_o[:] = _dot(_dot(xu_, Wdiu[:]), adiu[:])
    hsi = _dot(xi_, Wsiu[:])
    hsi_o[:] = hsi
    si_o[:] = _dot(hsi, aiu[:])
    dui_o[:] = _dot(_dot(xi_, Wdui[:]), adui[:])
    linu_o[:] = _dot(xu_, Wlu[:]) + blu[:]
    lini_o[:] = _dot(xi_, Wli[:]) + bli[:]


def _mid_body(aggi, aggu, lini, linu, b1ui, b1iu,
              Ws2ui, as2ui, Wd2ui, ad2ui, Ws2iu, as2iu, Wd2iu, ad2iu,
              hs2u_o, s2u_o, d2iu_o, hs2i_o, s2i_o, d2ui_o):
    hi = jnp.maximum(aggi[:] + b1ui[:] + lini[:], 0.0)
    hu = jnp.maximum(aggu[:] + b1iu[:] + linu[:], 0.0)
    hs2u = _dot(hu, Ws2ui[:])
    hs2u_o[:] = hs2u
    s2u_o[:] = _dot(hs2u, as2ui[:])
    d2iu_o[:] = _dot(_dot(hu, Wd2iu[:]), ad2iu[:])
    hs2i = _dot(hi, Ws2iu[:])
    hs2i_o[:] = hs2i
    s2i_o[:] = _dot(hs2i, as2iu[:])
    d2ui_o[:] = _dot(_dot(hi, Wd2ui[:]), ad2ui[:])


def _decpre_body(aggzu, aggzi, b2iu, b2ui, Wtop, Wbot, b1d, Zu_o, Zi_o):
    Zu_o[:] = _dot(aggzu[:] + b2iu[:], Wtop[:]) + b1d[:]
    Zi_o[:] = _dot(aggzi[:] + b2ui[:], Wbot[:])


def _decpost_body(He, w2, b2, out_o):
    h = jnp.maximum(He[:], 0.0)
    z = _dot(h, w2[:]) + b2[:]
    out_o[:] = jax.nn.sigmoid(z)


def _full(shape):
    return pl.BlockSpec(shape, lambda i: (0, 0))


def _rows(width):
    return pl.BlockSpec((TB, width), lambda i: (i, 0))


def _edge2_body(s0_h, d0_h, row0_h, col0_h,
                s1_h, d1_h, row1_h, col1_h,
                eraw0_h, eraw1_h, den_h,
                s_v, d_v, row_v, col_v, col2d_v, eraw_v, zbuf,
                den_sp0, den_sp1):
    cid = lax.axis_index("c")
    sid = lax.axis_index("s")
    wid = sid * 2 + cid

    def zb(k, c):
        zbuf[pl.ds(k * 16, 16)] = jnp.zeros((16,), f32)
        return c
    lax.fori_loop(0, NPAD // 16 // 16, zb, 0)
    dslc = pl.ds(pl.multiple_of(sid * (NPAD // 16), 8), NPAD // 16)
    pltpu.sync_copy(zbuf, den_sp0.at[dslc])
    pltpu.sync_copy(zbuf, den_sp1.at[dslc])
    plsc.subcore_barrier()

    ebase = wid * (EPAD // 32)
    for q, (s_h, d_h, row_h, col_h, eraw_h, den_sp) in enumerate([
            (s0_h, d0_h, row0_h, col0_h, eraw0_h, den_sp0),
            (s1_h, d1_h, row1_h, col1_h, eraw1_h, den_sp1)]):
        pltpu.sync_copy(s_h, s_v)
        pltpu.sync_copy(d_h, d_v)

        def chbody(ch, c0):
            cbase = pl.multiple_of(ebase + ch * 2048, 2048)
            pltpu.sync_copy(row_h.at[pl.ds(cbase, 2048)], row_v)
            pltpu.sync_copy(col_h.at[pl.ds(cbase, 2048)], col_v)

            def body(g, c):
                sl = pl.ds(g * 16, 16)
                rv = row_v[sl]
                cv = col_v[sl]
                sv = plsc.load_gather(s_v, [rv])
                dv = plsc.load_gather(d_v, [cv])
                a = sv + dv
                a = jnp.where(a > 0, a, 0.2 * a)
                e = jnp.exp(a)
                eid = cbase + g * 16 + lax.iota(i32, 16)
                e = jnp.where(eid < E_N, e, 0.0)
                eraw_v[sl] = e
                col2d_v[g // 8, pl.ds((g % 8) * 16, 16)] = cv
                return c
            lax.fori_loop(0, 128, body, 0)
            pltpu.sync_copy(eraw_v, eraw_h.at[pl.ds(cbase, 2048)])

            def kbody(k, c2):
                pltpu.sync_copy(
                    eraw_v.at[pl.ds(pl.multiple_of(k * 128, 128), 128)],
                    den_sp.at[col2d_v.at[k]], add=True)
                return c2
            lax.fori_loop(0, 16, kbody, 0)
            return c0
        lax.fori_loop(0, 5, chbody, 0)
    plsc.subcore_barrier()
    for q, den_sp in enumerate([den_sp0, den_sp1]):
        doff = pl.multiple_of((cid * 2 + q) * NPAD + sid * (NPAD // 16), 8)
        pltpu.sync_copy(den_sp.at[dslc], zbuf)
        pltpu.sync_copy(zbuf, den_h.at[pl.ds(doff, NPAD // 16)])


def _alpha2_body(row0_h, col0_h, eraw0_h, row1_h, col1_h, eraw1_h, den_h,
                 alpha0_h, alpha1_h, packed0_h, packed1_h,
                 den_v, den2_v, pack_v, e_v, a_v):
    cid = lax.axis_index("c")
    sid = lax.axis_index("s")
    wid = sid * 2 + cid
    ebase = wid * (EPAD // 32)
    for q, (row_h, col_h, eraw_h, alpha_h, packed_h) in enumerate([
            (row0_h, col0_h, eraw0_h, alpha0_h, packed0_h),
            (row1_h, col1_h, eraw1_h, alpha1_h, packed1_h)]):
        pltpu.sync_copy(den_h.at[pl.ds(q * NPAD, NPAD)], den_v)
        pltpu.sync_copy(den_h.at[pl.ds((2 + q) * NPAD, NPAD)], den2_v)

        def addb(k, c):
            sl = pl.ds(k * 16, 16)
            den_v[sl] = den_v[sl] + den2_v[sl]
            return c
        lax.fori_loop(0, NPAD // 16, addb, 0)

        def chbody(ch, c0):
            cbase = pl.multiple_of(ebase + ch * 2048, 2048)
            pltpu.sync_copy(row_h.at[pl.ds(cbase, 2048)],
                            pack_v.at[pl.ds(0, 2048)])
            pltpu.sync_copy(col_h.at[pl.ds(cbase, 2048)],
                            pack_v.at[pl.ds(2048, 2048)])
            pltpu.sync_copy(eraw_h.at[pl.ds(cbase, 2048)], e_v)

            def body(g, c):
                sl = pl.ds(g * 16, 16)
                cv = pack_v[pl.ds(2048 + g * 16, 16)]
                ev = e_v[sl]
                dv = plsc.load_gather(den_v, [cv])
                al = ev / (dv + 1e-16)
                a_v[sl] = al
                pack_v[pl.ds(4096 + g * 16, 16)] = plsc.bitcast(al, i32)
                return c
            lax.fori_loop(0, 128, body, 0)
            pltpu.sync_copy(a_v, alpha_h.at[pl.ds(cbase, 2048)])
            pltpu.sync_copy(pack_v,
                            packed_h.at[pl.ds(pl.multiple_of(cbase * 3,
                                                             2048), 6144)])
            return c0
        lax.fori_loop(0, 5, chbody, 0)


def _feat2_body(packed0_h, hs0_h, packed1_h, hs1_h,
                agg0_h, agg1_h,
                scan_v, comp_pack, comp_al,
                idx_row, idx_dst, idx_row2, idx_dst2, grows, grows2, zbuf,
                out_sp, sem, sem2):
    cid = lax.axis_index("c")
    sid = lax.axis_index("s")

    def zb(r, c):
        for k in range(8):
            zbuf[r, pl.ds(k * 16, 16)] = jnp.zeros((16,), f32)
        return c
    lax.fori_loop(0, 64, zb, 0)

    rb = pl.multiple_of(sid * FTR, 8)
    for q, (packed_h, hs_h, agg_h) in enumerate([
            (packed0_h, hs0_h, agg0_h),
            (packed1_h, hs1_h, agg1_h)]):

        def tbody(t, c9):
            ck = cid * 5 + t
            lo = pl.multiple_of(ck * FCH, 128)
            hi = lo + FCH

            for i in range(5):
                pltpu.sync_copy(
                    zbuf,
                    out_sp.at[pl.ds(pl.multiple_of(rb + i * 64, 8), 64)])
            plsc.subcore_barrier()

            sbase = sid * (EPAD // 16)

            def chbody(ch, ptr):
                cbase = pl.multiple_of(sbase + ch * 2048, 2048)
                pltpu.sync_copy(
                    packed_h.at[pl.ds(pl.multiple_of(cbase * 3, 2048),
                                      6144)], scan_v)

                def sbody(g2, ptr):
                    for g in (g2 * 2, g2 * 2 + 1):
                        rv = scan_v[pl.ds(g * 16, 16)]
                        cv = scan_v[pl.ds(2048 + g * 16, 16)]
                        av = plsc.bitcast(
                            scan_v[pl.ds(4096 + g * 16, 16)], f32)
                        m = (cv >= lo) & (cv < hi)
                        mi = jnp.where(m, 1, 0).astype(i32)
                        pk = rv + ((cv - lo) << 16)
                        psl = pl.ds(ptr, 16)
                        plsc.store_compressed(comp_pack.at[psl], pk, mask=m)
                        plsc.store_compressed(comp_al.at[psl], av, mask=m)
                        ptr = ptr + jnp.sum(mi)
                    return ptr
                return lax.fori_loop(0, 64, sbody, ptr)
            ptr = lax.fori_loop(0, 10, chbody, jnp.int32(0))

            cntp = ((ptr + 127) // 128) * 128
            zi16 = jnp.zeros((16,), i32)
            zf16 = jnp.zeros((16,), f32)

            def pbody(i, c):
                idxs = ptr + i * 16 + lax.iota(i32, 16)
                pm = idxs < cntp
                plsc.store_scatter(comp_pack, [idxs], zi16, mask=pm)
                plsc.store_scatter(comp_al, [idxs], zf16, mask=pm)
                return c
            lax.fori_loop(0, 8, pbody, 0)

            nb = cntp // 128

            def prep(off, idxr, idxd):
                def cp(i, c2):
                    s16 = pl.ds(off + i * 16, 16)
                    d16 = pl.ds(i * 16, 16)
                    pk = comp_pack[s16]
                    idxr[d16] = pk & 0xFFFF
                    idxd[d16] = pk >> 16
                    return c2
                lax.fori_loop(0, 8, cp, 0)

            def mul(off, g):
                def mul_r(r4, c2):
                    for r2 in range(4):
                        r = r4 * 4 + r2
                        av = plsc.load_gather(
                            comp_al, [jnp.full((16,), off + r, i32)])
                        for k in range(8):
                            sl = pl.ds(k * 16, 16)
                            g[r, sl] = g[r, sl] * av
                    return c2
                lax.fori_loop(0, 32, mul_r, 0)

            @pl.when(nb > 0)
            def _():
                prep(0, idx_row, idx_dst)
                pltpu.async_copy(hs_h.at[idx_row], grows, sem)

            def pair(i, c):
                b0 = i * 2
                pltpu.make_async_copy(hs_h.at[idx_row], grows, sem).wait()

                @pl.when(b0 + 1 < nb)
                def _():
                    prep((b0 + 1) * 128, idx_row2, idx_dst2)
                    pltpu.async_copy(hs_h.at[idx_row2], grows2, sem2)
                mul(b0 * 128, grows)
                pltpu.sync_copy(grows, out_sp.at[idx_dst], add=True)

                @pl.when(b0 + 1 < nb)
                def _():
                    pltpu.make_async_copy(hs_h.at[idx_row2], grows2,
                                          sem2).wait()

                    @pl.when(b0 + 2 < nb)
                    def _():
                        prep((b0 + 2) * 128, idx_row, idx_dst)
                        pltpu.async_copy(hs_h.at[idx_row], grows, sem)
                    mul((b0 + 1) * 128, grows2)
                    pltpu.sync_copy(grows2, out_sp.at[idx_dst2], add=True)
                return c
            lax.fori_loop(0, (nb + 1) // 2, pair, 0)
            plsc.subcore_barrier()
            for i in range(2):
                roff = pl.multiple_of(rb + i * 128, 8)
                pltpu.sync_copy(out_sp.at[pl.ds(roff, 128)], grows)
                pltpu.sync_copy(
                    grows, agg_h.at[pl.ds(pl.multiple_of(lo + roff, 8),
                                          128)])
            roff = pl.multiple_of(rb + 256, 8)
            pltpu.sync_copy(out_sp.at[pl.ds(roff, 64)],
                            grows.at[pl.ds(0, 64)])
            pltpu.sync_copy(
                grows.at[pl.ds(0, 64)],
                agg_h.at[pl.ds(pl.multiple_of(lo + roff, 8), 64)])
            plsc.subcore_barrier()
            return c9
        lax.fori_loop(0, 5, tbody, 0)


def _dec_body(rowl_h, coll_h, zu_h, zi_h, he_h,
              idx_u, idx_i, idx_u2, idx_i2, gu, gi, gu2, gi2, sem, sem2):
    cid = lax.axis_index("c")
    sid = lax.axis_index("s")
    wid = sid * 2 + cid
    base = wid * (ELPAD // 32)
    nb = ELPAD // 32 // 128

    def start(b, iu, ii, bu, bi, s):
        off = pl.multiple_of(base + b * 128, 128)
        pltpu.sync_copy(rowl_h.at[pl.ds(off, 128)], iu)
        pltpu.sync_copy(coll_h.at[pl.ds(off, 128)], ii)
        pltpu.async_copy(zu_h.at[iu], bu, s)
        pltpu.async_copy(zi_h.at[ii], bi, s)

    def finish(b, iu, ii, bu, bi, s):
        off = pl.multiple_of(base + b * 128, 128)
        pltpu.make_async_copy(zu_h.at[iu], bu, s).wait()
        pltpu.make_async_copy(zi_h.at[ii], bi, s).wait()

        def addr(r4, c2):
            for r2 in range(4):
                r = r4 * 4 + r2
                for k in range(8):
                    sl = pl.ds(k * 16, 16)
                    bu[r, sl] = bu[r, sl] + bi[r, sl]
            return c2
        lax.fori_loop(0, 32, addr, 0)
        pltpu.sync_copy(bu, he_h.at[pl.ds(off, 128)])

    start(0, idx_u, idx_i, gu, gi, sem)

    def body(p, c):
        b0 = p * 2
        start(b0 + 1, idx_u2, idx_i2, gu2, gi2, sem2)
        finish(b0, idx_u, idx_i, gu, gi, sem)

        @pl.when(b0 + 2 < nb)
        def _():
            start(b0 + 2, idx_u, idx_i, gu, gi, sem)
        finish(b0 + 1, idx_u2, idx_i2, gu2, gi2, sem2)
        return c
    lax.fori_loop(0, nb // 2, body, 0)


def _mesh():
    return plsc.VectorSubcoreMesh(core_axis_name="c", subcore_axis_name="s")


_SC_PARAMS = pltpu.CompilerParams(needs_layout_passes=False)


def _edge2_call(s0, d0, row0, col0, s1, d1, row1, col1):
    return pl.kernel(
        _edge2_body,
        out_type=[jax.ShapeDtypeStruct((EPAD,), f32),
                  jax.ShapeDtypeStruct((EPAD,), f32),
                  jax.ShapeDtypeStruct((4 * NPAD,), f32)],
        mesh=_mesh(),
        compiler_params=_SC_PARAMS,
        scratch_types=[
            pltpu.VMEM((N,), f32),
            pltpu.VMEM((N,), f32),
            pltpu.VMEM((2048,), i32),
            pltpu.VMEM((2048,), i32),
            pltpu.VMEM((16, 128), i32),
            pltpu.VMEM((2048,), f32),
            pltpu.VMEM((NPAD // 16,), f32),
            pltpu.VMEM_SHARED((NPAD,), f32),
            pltpu.VMEM_SHARED((NPAD,), f32),
        ],
    )(s0, d0, row0, col0, s1, d1, row1, col1)


def _alpha2_call(row0, col0, eraw0, row1, col1, eraw1, den4):
    return pl.kernel(
        _alpha2_body,
        out_type=[jax.ShapeDtypeStruct((EPAD,), f32),
                  jax.ShapeDtypeStruct((EPAD,), f32),
                  jax.ShapeDtypeStruct((3 * EPAD,), i32),
                  jax.ShapeDtypeStruct((3 * EPAD,), i32)],
        mesh=_mesh(),
        compiler_params=_SC_PARAMS,
        scratch_types=[
            pltpu.VMEM((NPAD,), f32),
            pltpu.VMEM((NPAD,), f32),
            pltpu.VMEM((6144,), i32),
            pltpu.VMEM((2048,), f32),
            pltpu.VMEM((2048,), f32),
        ],
    )(row0, col0, eraw0, row1, col1, eraw1, den4)


def _feat2_call(packed0, hs0, packed1, hs1):
    return pl.kernel(
        _feat2_body,
        out_type=[jax.ShapeDtypeStruct((FAGG, H), f32),
                  jax.ShapeDtypeStruct((FAGG, H), f32)],
        mesh=_mesh(),
        compiler_params=_SC_PARAMS,
        scratch_types=[
            pltpu.VMEM((6144,), i32),
            pltpu.VMEM((20608,), i32),
            pltpu.VMEM((20608,), f32),
            pltpu.VMEM((128,), i32),
            pltpu.VMEM((128,), i32),
            pltpu.VMEM((128,), i32),
            pltpu.VMEM((128,), i32),
            pltpu.VMEM((128, H), f32),
            pltpu.VMEM((128, H), f32),
            pltpu.VMEM((64, H), f32),
            pltpu.VMEM_SHARED((FCH, H), f32),
            pltpu.SemaphoreType.DMA,
            pltpu.SemaphoreType.DMA,
        ],
    )(packed0, hs0, packed1, hs1)


def _dec_call(rowl, coll, Zu, Zi):
    return pl.kernel(
        _dec_body,
        out_type=[jax.ShapeDtypeStruct((ELPAD, H), f32)],
        mesh=_mesh(),
        compiler_params=_SC_PARAMS,
        scratch_types=[
            pltpu.VMEM((128,), i32),
            pltpu.VMEM((128,), i32),
            pltpu.VMEM((128,), i32),
            pltpu.VMEM((128,), i32),
            pltpu.VMEM((128, H), f32),
            pltpu.VMEM((128, H), f32),
            pltpu.VMEM((128, H), f32),
            pltpu.VMEM((128, H), f32),
            pltpu.SemaphoreType.DMA,
            pltpu.SemaphoreType.DMA,
        ],
    )(rowl, coll, Zu, Zi)[0]


def _run_layer(s0, d0, hs0, s1, d1, hs1,
               row0, col0, row1, col1):
    eraw0, eraw1, den4 = _edge2_call(s0, d0, row0, col0,
                                     s1, d1, row1, col1)
    alpha0, alpha1, packed0, packed1 = _alpha2_call(
        row0, col0, eraw0, row1, col1, eraw1, den4)
    agg0, agg1 = _feat2_call(packed0, hs0, packed1, hs1)
    return agg0, agg1, alpha0, alpha1


def kernel(x_user, x_item, edge_index_ui, edge_index_iu, edge_label_index,
           params):
    p = params
    def padE(x):
        return jnp.pad(x.astype(i32), (0, EPAD - E_N), mode='wrap')

    def padL(x):
        return jnp.pad(x.astype(i32), (0, ELPAD - EL_N), mode='wrap')

    row_ui = padE(edge_index_ui[0])
    col_ui = padE(edge_index_ui[1])
    row_iu = padE(edge_index_iu[0])
    col_iu = padE(edge_index_iu[1])
    rowl = padL(edge_label_index[0])
    coll = padL(edge_label_index[1])

    c1ui, c1iu = p['conv1_ui'], p['conv1_iu']
    c2ui, c2iu = p['conv2_ui'], p['conv2_iu']
    lu, li = p['lin1_user'], p['lin1_item']

    nt = N // TB
    v = lambda x: x.reshape(H, 1)
    b = lambda x: x.reshape(1, H)
    sds = jax.ShapeDtypeStruct

    (hsu, su1, diu1, hsi, si1, dui1, linu, lini) = pl.pallas_call(
        _pre1_body,
        grid=(nt,),
        in_specs=[_rows(H), _rows(H)] + [_full((H, H)), _full((H, 1))] * 4
        + [_full((H, H)), _full((1, H))] * 2,
        out_specs=[_rows(H), _rows(1), _rows(1), _rows(H), _rows(1),
                   _rows(1), _rows(H), _rows(H)],
        out_shape=[sds((N, H), f32), sds((N, 1), f32), sds((N, 1), f32),
                   sds((N, H), f32), sds((N, 1), f32), sds((N, 1), f32),
                   sds((N, H), f32), sds((N, H), f32)],
    )(x_user, x_item,
      c1ui['Ws'], v(c1ui['as']), c1ui['Wd'], v(c1ui['ad']),
      c1iu['Ws'], v(c1iu['as']), c1iu['Wd'], v(c1iu['ad']),
      lu['W'], b(lu['b']), li['W'], b(li['b']))

    agg_i1, agg_u1, _, _ = _run_layer(
        su1.reshape(-1), dui1.reshape(-1), hsu,
        si1.reshape(-1), diu1.reshape(-1), hsi,
        row_ui, col_ui, row_iu, col_iu)

    (hs2u, s2u, d2iu, hs2i, s2i, d2ui) = pl.pallas_call(
        _mid_body,
        grid=(nt,),
        in_specs=[_rows(H)] * 4 + [_full((1, H))] * 2
        + [_full((H, H)), _full((H, 1))] * 4,
        out_specs=[_rows(H), _rows(1), _rows(1), _rows(H), _rows(1),
                   _rows(1)],
        out_shape=[sds((N, H), f32), sds((N, 1), f32), sds((N, 1), f32),
                   sds((N, H), f32), sds((N, 1), f32), sds((N, 1), f32)],
    )(agg_i1[:N], agg_u1[:N], lini, linu, b(c1ui['b']), b(c1iu['b']),
      c2ui['Ws'], v(c2ui['as']), c2ui['Wd'], v(c2ui['ad']),
      c2iu['Ws'], v(c2iu['as']), c2iu['Wd'], v(c2iu['ad']))

    agg_zi, agg_zu, alpha_ui, alpha_iu = _run_layer(
        s2u.reshape(-1), d2ui.reshape(-1), hs2u,
        s2i.reshape(-1), d2iu.reshape(-1), hs2i,
        row_ui, col_ui, row_iu, col_iu)

    Wd1 = p['dec1']['W']
    Zu, Zi = pl.pallas_call(
        _decpre_body,
        grid=(nt,),
        in_specs=[_rows(H), _rows(H), _full((1, H)), _full((1, H)),
                  _full((H, H)), _full((H, H)), _full((1, H))],
        out_specs=[_rows(H), _rows(H)],
        out_shape=[sds((N, H), f32), sds((N, H), f32)],
    )(agg_zu[:N], agg_zi[:N], b(c2iu['b']), b(c2ui['b']),
      Wd1[:H], Wd1[H:], b(p['dec1']['b']))

    He = _dec_call(rowl, coll, Zu, Zi)

    predp = pl.pallas_call(
        _decpost_body,
        grid=(ELPAD // 1024,),
        in_specs=[pl.BlockSpec((1024, H), lambda i: (i, 0)),
                  _full((H, 1)), _full((1, 1))],
        out_specs=pl.BlockSpec((1024, 1), lambda i: (i, 0)),
        out_shape=sds((ELPAD, 1), f32),
    )(He, p['dec2']['W'], p['dec2']['b'].reshape(1, 1))

    pred = predp[:EL_N, 0]
    return pred, alpha_ui[:E_N], alpha_iu[:E_N]


# speedup vs baseline: 1.3106x; 1.0063x over previous
import jax
import jax.numpy as jnp
from jax import lax
from jax.experimental import pallas as pl
from jax.experimental.pallas import tpu as pltpu
from jax.experimental.pallas import tpu_sc as plsc

H = 128
HH = 64
N = 50000
E_N = 300000
EL_N = 200000
EPAD = 327680
ELPAD = 204800
NPAD = 50176
FCH = 5120
FAGG = 51200
FTR = FCH // 16
TB = 1000

f32 = jnp.float32
i32 = jnp.int32


def _dot(a, b):
    return jnp.dot(a, b, preferred_element_type=f32)


def _pre1_body(xu, xi, Wsui, aui, Wdui, adui, Wsiu, aiu, Wdiu, adiu,
               Wlu, blu, Wli, bli,
               hsu_o, su_o, diu_o, hsi_o, si_o, dui_o,
               linu_o, lini_o):
    xu_ = xu[:]
    xi_ = xi[:]
    hsu = _dot(xu_, Wsui[:])
    hsu_o[:] = hsu
    su_o[:] = _dot(hsu, aui[:])
    diu_o[:] = _dot(_dot(xu_, Wdiu[:]), adiu[:])
    hsi = _dot(xi_, Wsiu[:])
    hsi_o[:] = hsi
    si_o[:] = _dot(hsi, aiu[:])
    dui_o[:] = _dot(_dot(xi_, Wdui[:]), adui[:])
    linu_o[:] = _dot(xu_, Wlu[:]) + blu[:]
    lini_o[:] = _dot(xi_, Wli[:]) + bli[:]


def _mid_body(aggi, aggu, lini, linu, b1ui, b1iu,
              Ws2ui, as2ui, Wd2ui, ad2ui, Ws2iu, as2iu, Wd2iu, ad2iu,
              hs2u_o, s2u_o, d2iu_o, hs2i_o, s2i_o, d2ui_o):
    hi = jnp.maximum(aggi[:] + b1ui[:] + lini[:], 0.0)
    hu = jnp.maximum(aggu[:] + b1iu[:] + linu[:], 0.0)
    hs2u = _dot(hu, Ws2ui[:])
    hs2u_o[:] = hs2u
    s2u_o[:] = _dot(hs2u, as2ui[:])
    d2iu_o[:] = _dot(_dot(hu, Wd2iu[:]), ad2iu[:])
    hs2i = _dot(hi, Ws2iu[:])
    hs2i_o[:] = hs2i
    s2i_o[:] = _dot(hs2i, as2iu[:])
    d2ui_o[:] = _dot(_dot(hi, Wd2ui[:]), ad2ui[:])


def _decpre_body(aggzu, aggzi, b2iu, b2ui, Wtop, Wbot, b1d, Zu_o, Zi_o):
    Zu_o[:] = _dot(aggzu[:] + b2iu[:], Wtop[:]) + b1d[:]
    Zi_o[:] = _dot(aggzi[:] + b2ui[:], Wbot[:])


def _decpost_body(He, w2, b2, out_o):
    h = jnp.maximum(He[:], 0.0)
    z = _dot(h, w2[:]) + b2[:]
    out_o[:] = jax.nn.sigmoid(z)


def _full(shape):
    return pl.BlockSpec(shape, lambda i: (0, 0))


def _rows(width):
    return pl.BlockSpec((TB, width), lambda i: (i, 0))


def _edge2_body(s0_h, d0_h, row0_h, col0_h,
                s1_h, d1_h, row1_h, col1_h,
                eraw0_h, eraw1_h, den_h,
                s_v, d_v, row_v, col_v, col2d_v, eraw_v, zbuf,
                den_sp0, den_sp1):
    cid = lax.axis_index("c")
    sid = lax.axis_index("s")
    wid = sid * 2 + cid

    def zb(k, c):
        zbuf[pl.ds(k * 16, 16)] = jnp.zeros((16,), f32)
        return c
    lax.fori_loop(0, NPAD // 16 // 16, zb, 0)
    dslc = pl.ds(pl.multiple_of(sid * (NPAD // 16), 8), NPAD // 16)
    pltpu.sync_copy(zbuf, den_sp0.at[dslc])
    pltpu.sync_copy(zbuf, den_sp1.at[dslc])
    plsc.subcore_barrier()

    ebase = wid * (EPAD // 32)
    for q, (s_h, d_h, row_h, col_h, eraw_h, den_sp) in enumerate([
            (s0_h, d0_h, row0_h, col0_h, eraw0_h, den_sp0),
            (s1_h, d1_h, row1_h, col1_h, eraw1_h, den_sp1)]):
        pltpu.sync_copy(s_h, s_v)
        pltpu.sync_copy(d_h, d_v)

        def chbody(ch, c0):
            cbase = pl.multiple_of(ebase + ch * 2048, 2048)
            pltpu.sync_copy(row_h.at[pl.ds(cbase, 2048)], row_v)
            pltpu.sync_copy(col_h.at[pl.ds(cbase, 2048)], col_v)

            def body(g, c):
                sl = pl.ds(g * 16, 16)
                rv = row_v[sl]
                cv = col_v[sl]
                sv = plsc.load_gather(s_v, [rv])
                dv = plsc.load_gather(d_v, [cv])
                a = sv + dv
                a = jnp.where(a > 0, a, 0.2 * a)
                e = jnp.exp(a)
                eid = cbase + g * 16 + lax.iota(i32, 16)
                e = jnp.where(eid < E_N, e, 0.0)
                eraw_v[sl] = e
                col2d_v[g // 8, pl.ds((g % 8) * 16, 16)] = cv
                return c
            lax.fori_loop(0, 128, body, 0)
            pltpu.sync_copy(eraw_v, eraw_h.at[pl.ds(cbase, 2048)])

            def kbody(k, c2):
                pltpu.sync_copy(
                    eraw_v.at[pl.ds(pl.multiple_of(k * 128, 128), 128)],
                    den_sp.at[col2d_v.at[k]], add=True)
                return c2
            lax.fori_loop(0, 16, kbody, 0)
            return c0
        lax.fori_loop(0, 5, chbody, 0)
    plsc.subcore_barrier()
    for q, den_sp in enumerate([den_sp0, den_sp1]):
        doff = pl.multiple_of((cid * 2 + q) * NPAD + sid * (NPAD // 16), 8)
        pltpu.sync_copy(den_sp.at[dslc], zbuf)
        pltpu.sync_copy(zbuf, den_h.at[pl.ds(doff, NPAD // 16)])


def _alpha2_body(row0_h, col0_h, eraw0_h, row1_h, col1_h, eraw1_h, den_h,
                 alpha0_h, alpha1_h, packed0_h, packed1_h,
                 den_v, den2_v, pack_v, e_v, a_v):
    cid = lax.axis_index("c")
    sid = lax.axis_index("s")
    wid = sid * 2 + cid
    ebase = wid * (EPAD // 32)
    for q, (row_h, col_h, eraw_h, alpha_h, packed_h) in enumerate([
            (row0_h, col0_h, eraw0_h, alpha0_h, packed0_h),
            (row1_h, col1_h, eraw1_h, alpha1_h, packed1_h)]):
        pltpu.sync_copy(den_h.at[pl.ds(q * NPAD, NPAD)], den_v)
        pltpu.sync_copy(den_h.at[pl.ds((2 + q) * NPAD, NPAD)], den2_v)

        def addb(k, c):
            sl = pl.ds(k * 16, 16)
            den_v[sl] = den_v[sl] + den2_v[sl]
            return c
        lax.fori_loop(0, NPAD // 16, addb, 0)

        def chbody(ch, c0):
            cbase = pl.multiple_of(ebase + ch * 2048, 2048)
            pltpu.sync_copy(row_h.at[pl.ds(cbase, 2048)],
                            pack_v.at[pl.ds(0, 2048)])
            pltpu.sync_copy(col_h.at[pl.ds(cbase, 2048)],
                            pack_v.at[pl.ds(2048, 2048)])
            pltpu.sync_copy(eraw_h.at[pl.ds(cbase, 2048)], e_v)

            def body(g, c):
                sl = pl.ds(g * 16, 16)
                cv = pack_v[pl.ds(2048 + g * 16, 16)]
                ev = e_v[sl]
                dv = plsc.load_gather(den_v, [cv])
                al = ev / (dv + 1e-16)
                a_v[sl] = al
                pack_v[pl.ds(4096 + g * 16, 16)] = plsc.bitcast(al, i32)
                return c
            lax.fori_loop(0, 128, body, 0)
            pltpu.sync_copy(a_v, alpha_h.at[pl.ds(cbase, 2048)])
            pltpu.sync_copy(pack_v,
                            packed_h.at[pl.ds(pl.multiple_of(cbase * 3,
                                                             2048), 6144)])
            return c0
        lax.fori_loop(0, 5, chbody, 0)


def _feat2_body(packed0_h, hs0_h, packed1_h, hs1_h,
                agg0_h, agg1_h,
                scan_v, comp_pack, comp_al,
                idx_row, idx_dst, idx_row2, idx_dst2, grows, grows2, zbuf,
                out_sp, sem, sem2, sems, sems2):
    cid = lax.axis_index("c")
    sid = lax.axis_index("s")

    def zb(r, c):
        for k in range(8):
            zbuf[r, pl.ds(k * 16, 16)] = jnp.zeros((16,), f32)
        return c
    lax.fori_loop(0, 64, zb, 0)

    rb = pl.multiple_of(sid * FTR, 8)
    for q, (packed_h, hs_h, agg_h) in enumerate([
            (packed0_h, hs0_h, agg0_h),
            (packed1_h, hs1_h, agg1_h)]):

        def tbody(t, c9):
            ck = cid * 5 + t
            lo = pl.multiple_of(ck * FCH, 128)
            hi = lo + FCH

            for i in range(5):
                pltpu.sync_copy(
                    zbuf,
                    out_sp.at[pl.ds(pl.multiple_of(rb + i * 64, 8), 64)])
            plsc.subcore_barrier()

            sbase = sid * (EPAD // 16)

            def chbody(ch, ptr):
                cbase = pl.multiple_of(sbase + ch * 2048, 2048)
                pltpu.sync_copy(
                    packed_h.at[pl.ds(pl.multiple_of(cbase * 3, 2048),
                                      6144)], scan_v)

                def sbody(g2, ptr):
                    for g in (g2 * 2, g2 * 2 + 1):
                        rv = scan_v[pl.ds(g * 16, 16)]
                        cv = scan_v[pl.ds(2048 + g * 16, 16)]
                        av = plsc.bitcast(
                            scan_v[pl.ds(4096 + g * 16, 16)], f32)
                        m = (cv >= lo) & (cv < hi)
                        mi = jnp.where(m, 1, 0).astype(i32)
                        pk = rv + ((cv - lo) << 16)
                        psl = pl.ds(ptr, 16)
                        plsc.store_compressed(comp_pack.at[psl], pk, mask=m)
                        plsc.store_compressed(comp_al.at[psl], av, mask=m)
                        ptr = ptr + jnp.sum(mi)
                    return ptr
                return lax.fori_loop(0, 64, sbody, ptr)
            ptr = lax.fori_loop(0, 10, chbody, jnp.int32(0))

            cntp = ((ptr + 127) // 128) * 128
            zi16 = jnp.zeros((16,), i32)
            zf16 = jnp.zeros((16,), f32)

            def pbody(i, c):
                idxs = ptr + i * 16 + lax.iota(i32, 16)
                pm = idxs < cntp
                plsc.store_scatter(comp_pack, [idxs], zi16, mask=pm)
                plsc.store_scatter(comp_al, [idxs], zf16, mask=pm)
                return c
            lax.fori_loop(0, 8, pbody, 0)

            nb = cntp // 128

            def prep(off, idxr, idxd):
                def cp(i, c2):
                    s16 = pl.ds(off + i * 16, 16)
                    d16 = pl.ds(i * 16, 16)
                    pk = comp_pack[s16]
                    idxr[d16] = pk & 0xFFFF
                    idxd[d16] = pk >> 16
                    return c2
                lax.fori_loop(0, 8, cp, 0)

            def mul(off, g):
                def mul_r(r4, c2):
                    for r2 in range(4):
                        r = r4 * 4 + r2
                        av = plsc.load_gather(
                            comp_al, [jnp.full((16,), off + r, i32)])
                        for k in range(8):
                            sl = pl.ds(k * 16, 16)
                            g[r, sl] = g[r, sl] * av
                    return c2
                lax.fori_loop(0, 32, mul_r, 0)

            @pl.when(nb > 0)
            def _():
                prep(0, idx_row, idx_dst)
                pltpu.async_copy(hs_h.at[idx_row], grows, sem)

            def pair(i, c):
                b0 = i * 2
                pltpu.make_async_copy(hs_h.at[idx_row], grows, sem).wait()

                @pl.when(b0 + 1 < nb)
                def _():
                    @pl.when(i > 0)
                    def _():
                        pltpu.make_async_copy(
                            grows2, out_sp.at[idx_dst2], sems2).wait()
                    prep((b0 + 1) * 128, idx_row2, idx_dst2)
                    pltpu.async_copy(hs_h.at[idx_row2], grows2, sem2)
                mul(b0 * 128, grows)
                pltpu.async_copy(grows, out_sp.at[idx_dst], sems, add=True)

                @pl.when(b0 + 1 < nb)
                def _():
                    pltpu.make_async_copy(hs_h.at[idx_row2], grows2,
                                          sem2).wait()

                    @pl.when(b0 + 2 < nb)
                    def _():
                        pltpu.make_async_copy(
                            grows, out_sp.at[idx_dst], sems).wait()
                        prep((b0 + 2) * 128, idx_row, idx_dst)
                        pltpu.async_copy(hs_h.at[idx_row], grows, sem)
                    mul((b0 + 1) * 128, grows2)
                    pltpu.async_copy(grows2, out_sp.at[idx_dst2], sems2,
                                     add=True)
                return c
            lax.fori_loop(0, (nb + 1) // 2, pair, 0)

            @pl.when(nb > 0)
            def _():
                pltpu.make_async_copy(grows, out_sp.at[idx_dst],
                                      sems).wait()

            @pl.when(nb > 1)
            def _():
                pltpu.make_async_copy(grows2, out_sp.at[idx_dst2],
                                      sems2).wait()
            plsc.subcore_barrier()
            for i in range(2):
                roff = pl.multiple_of(rb + i * 128, 8)
                pltpu.sync_copy(out_sp.at[pl.ds(roff, 128)], grows)
                pltpu.sync_copy(
                    grows, agg_h.at[pl.ds(pl.multiple_of(lo + roff, 8),
                                          128)])
            roff = pl.multiple_of(rb + 256, 8)
            pltpu.sync_copy(out_sp.at[pl.ds(roff, 64)],
                            grows.at[pl.ds(0, 64)])
            pltpu.sync_copy(
                grows.at[pl.ds(0, 64)],
                agg_h.at[pl.ds(pl.multiple_of(lo + roff, 8), 64)])
            plsc.subcore_barrier()
            return c9
        lax.fori_loop(0, 5, tbody, 0)


def _dec_body(rowl_h, coll_h, zu_h, zi_h, he_h,
              idx_u, idx_i, idx_u2, idx_i2, gu, gi, gu2, gi2, sem, sem2):
    cid = lax.axis_index("c")
    sid = lax.axis_index("s")
    wid = sid * 2 + cid
    base = wid * (ELPAD // 32)
    nb = ELPAD // 32 // 128

    def start(b, iu, ii, bu, bi, s):
        off = pl.multiple_of(base + b * 128, 128)
        pltpu.sync_copy(rowl_h.at[pl.ds(off, 128)], iu)
        pltpu.sync_copy(coll_h.at[pl.ds(off, 128)], ii)
        pltpu.async_copy(zu_h.at[iu], bu, s)
        pltpu.async_copy(zi_h.at[ii], bi, s)

    def finish(b, iu, ii, bu, bi, s):
        off = pl.multiple_of(base + b * 128, 128)
        pltpu.make_async_copy(zu_h.at[iu], bu, s).wait()
        pltpu.make_async_copy(zi_h.at[ii], bi, s).wait()

        def addr(r4, c2):
            for r2 in range(4):
                r = r4 * 4 + r2
                for k in range(8):
                    sl = pl.ds(k * 16, 16)
                    bu[r, sl] = bu[r, sl] + bi[r, sl]
            return c2
        lax.fori_loop(0, 32, addr, 0)
        pltpu.sync_copy(bu, he_h.at[pl.ds(off, 128)])

    start(0, idx_u, idx_i, gu, gi, sem)

    def body(p, c):
        b0 = p * 2
        start(b0 + 1, idx_u2, idx_i2, gu2, gi2, sem2)
        finish(b0, idx_u, idx_i, gu, gi, sem)

        @pl.when(b0 + 2 < nb)
        def _():
            start(b0 + 2, idx_u, idx_i, gu, gi, sem)
        finish(b0 + 1, idx_u2, idx_i2, gu2, gi2, sem2)
        return c
    lax.fori_loop(0, nb // 2, body, 0)


def _mesh():
    return plsc.VectorSubcoreMesh(core_axis_name="c", subcore_axis_name="s")


_SC_PARAMS = pltpu.CompilerParams(needs_layout_passes=False)


def _edge2_call(s0, d0, row0, col0, s1, d1, row1, col1):
    return pl.kernel(
        _edge2_body,
        out_type=[jax.ShapeDtypeStruct((EPAD,), f32),
                  jax.ShapeDtypeStruct((EPAD,), f32),
                  jax.ShapeDtypeStruct((4 * NPAD,), f32)],
        mesh=_mesh(),
        compiler_params=_SC_PARAMS,
        scratch_types=[
            pltpu.VMEM((N,), f32),
            pltpu.VMEM((N,), f32),
            pltpu.VMEM((2048,), i32),
            pltpu.VMEM((2048,), i32),
            pltpu.VMEM((16, 128), i32),
            pltpu.VMEM((2048,), f32),
            pltpu.VMEM((NPAD // 16,), f32),
            pltpu.VMEM_SHARED((NPAD,), f32),
            pltpu.VMEM_SHARED((NPAD,), f32),
        ],
    )(s0, d0, row0, col0, s1, d1, row1, col1)


def _alpha2_call(row0, col0, eraw0, row1, col1, eraw1, den4):
    return pl.kernel(
        _alpha2_body,
        out_type=[jax.ShapeDtypeStruct((EPAD,), f32),
                  jax.ShapeDtypeStruct((EPAD,), f32),
                  jax.ShapeDtypeStruct((3 * EPAD,), i32),
                  jax.ShapeDtypeStruct((3 * EPAD,), i32)],
        mesh=_mesh(),
        compiler_params=_SC_PARAMS,
        scratch_types=[
            pltpu.VMEM((NPAD,), f32),
            pltpu.VMEM((NPAD,), f32),
            pltpu.VMEM((6144,), i32),
            pltpu.VMEM((2048,), f32),
            pltpu.VMEM((2048,), f32),
        ],
    )(row0, col0, eraw0, row1, col1, eraw1, den4)


def _feat2_call(packed0, hs0, packed1, hs1):
    return pl.kernel(
        _feat2_body,
        out_type=[jax.ShapeDtypeStruct((FAGG, H), f32),
                  jax.ShapeDtypeStruct((FAGG, H), f32)],
        mesh=_mesh(),
        compiler_params=_SC_PARAMS,
        scratch_types=[
            pltpu.VMEM((6144,), i32),
            pltpu.VMEM((20608,), i32),
            pltpu.VMEM((20608,), f32),
            pltpu.VMEM((128,), i32),
            pltpu.VMEM((128,), i32),
            pltpu.VMEM((128,), i32),
            pltpu.VMEM((128,), i32),
            pltpu.VMEM((128, H), f32),
            pltpu.VMEM((128, H), f32),
            pltpu.VMEM((64, H), f32),
            pltpu.VMEM_SHARED((FCH, H), f32),
            pltpu.SemaphoreType.DMA,
            pltpu.SemaphoreType.DMA,
            pltpu.SemaphoreType.DMA,
            pltpu.SemaphoreType.DMA,
        ],
    )(packed0, hs0, packed1, hs1)


def _dec_call(rowl, coll, Zu, Zi):
    return pl.kernel(
        _dec_body,
        out_type=[jax.ShapeDtypeStruct((ELPAD, H), f32)],
        mesh=_mesh(),
        compiler_params=_SC_PARAMS,
        scratch_types=[
            pltpu.VMEM((128,), i32),
            pltpu.VMEM((128,), i32),
            pltpu.VMEM((128,), i32),
            pltpu.VMEM((128,), i32),
            pltpu.VMEM((128, H), f32),
            pltpu.VMEM((128, H), f32),
            pltpu.VMEM((128, H), f32),
            pltpu.VMEM((128, H), f32),
            pltpu.SemaphoreType.DMA,
            pltpu.SemaphoreType.DMA,
        ],
    )(rowl, coll, Zu, Zi)[0]


def _run_layer(s0, d0, hs0, s1, d1, hs1,
               row0, col0, row1, col1):
    eraw0, eraw1, den4 = _edge2_call(s0, d0, row0, col0,
                                     s1, d1, row1, col1)
    alpha0, alpha1, packed0, packed1 = _alpha2_call(
        row0, col0, eraw0, row1, col1, eraw1, den4)
    agg0, agg1 = _feat2_call(packed0, hs0, packed1, hs1)
    return agg0, agg1, alpha0, alpha1


def kernel(x_user, x_item, edge_index_ui, edge_index_iu, edge_label_index,
           params):
    p = params
    def padE(x):
        return jnp.pad(x.astype(i32), (0, EPAD - E_N), mode='wrap')

    def padL(x):
        return jnp.pad(x.astype(i32), (0, ELPAD - EL_N), mode='wrap')

    row_ui = padE(edge_index_ui[0])
    col_ui = padE(edge_index_ui[1])
    row_iu = padE(edge_index_iu[0])
    col_iu = padE(edge_index_iu[1])
    rowl = padL(edge_label_index[0])
    coll = padL(edge_label_index[1])

    c1ui, c1iu = p['conv1_ui'], p['conv1_iu']
    c2ui, c2iu = p['conv2_ui'], p['conv2_iu']
    lu, li = p['lin1_user'], p['lin1_item']

    nt = N // TB
    v = lambda x: x.reshape(H, 1)
    b = lambda x: x.reshape(1, H)
    sds = jax.ShapeDtypeStruct

    (hsu, su1, diu1, hsi, si1, dui1, linu, lini) = pl.pallas_call(
        _pre1_body,
        grid=(nt,),
        in_specs=[_rows(H), _rows(H)] + [_full((H, H)), _full((H, 1))] * 4
        + [_full((H, H)), _full((1, H))] * 2,
        out_specs=[_rows(H), _rows(1), _rows(1), _rows(H), _rows(1),
                   _rows(1), _rows(H), _rows(H)],
        out_shape=[sds((N, H), f32), sds((N, 1), f32), sds((N, 1), f32),
                   sds((N, H), f32), sds((N, 1), f32), sds((N, 1), f32),
                   sds((N, H), f32), sds((N, H), f32)],
    )(x_user, x_item,
      c1ui['Ws'], v(c1ui['as']), c1ui['Wd'], v(c1ui['ad']),
      c1iu['Ws'], v(c1iu['as']), c1iu['Wd'], v(c1iu['ad']),
      lu['W'], b(lu['b']), li['W'], b(li['b']))

    agg_i1, agg_u1, _, _ = _run_layer(
        su1.reshape(-1), dui1.reshape(-1), hsu,
        si1.reshape(-1), diu1.reshape(-1), hsi,
        row_ui, col_ui, row_iu, col_iu)

    (hs2u, s2u, d2iu, hs2i, s2i, d2ui) = pl.pallas_call(
        _mid_body,
        grid=(nt,),
        in_specs=[_rows(H)] * 4 + [_full((1, H))] * 2
        + [_full((H, H)), _full((H, 1))] * 4,
        out_specs=[_rows(H), _rows(1), _rows(1), _rows(H), _rows(1),
                   _rows(1)],
        out_shape=[sds((N, H), f32), sds((N, 1), f32), sds((N, 1), f32),
                   sds((N, H), f32), sds((N, 1), f32), sds((N, 1), f32)],
    )(agg_i1[:N], agg_u1[:N], lini, linu, b(c1ui['b']), b(c1iu['b']),
      c2ui['Ws'], v(c2ui['as']), c2ui['Wd'], v(c2ui['ad']),
      c2iu['Ws'], v(c2iu['as']), c2iu['Wd'], v(c2iu['ad']))

    agg_zi, agg_zu, alpha_ui, alpha_iu = _run_layer(
        s2u.reshape(-1), d2ui.reshape(-1), hs2u,
        s2i.reshape(-1), d2iu.reshape(-1), hs2i,
        row_ui, col_ui, row_iu, col_iu)

    Wd1 = p['dec1']['W']
    Zu, Zi = pl.pallas_call(
        _decpre_body,
        grid=(nt,),
        in_specs=[_rows(H), _rows(H), _full((1, H)), _full((1, H)),
                  _full((H, H)), _full((H, H)), _full((1, H))],
        out_specs=[_rows(H), _rows(H)],
        out_shape=[sds((N, H), f32), sds((N, H), f32)],
    )(agg_zu[:N], agg_zi[:N], b(c2iu['b']), b(c2ui['b']),
      Wd1[:H], Wd1[H:], b(p['dec1']['b']))

    He = _dec_call(rowl, coll, Zu, Zi)

    predp = pl.pallas_call(
        _decpost_body,
        grid=(ELPAD // 1024,),
        in_specs=[pl.BlockSpec((1024, H), lambda i: (i, 0)),
                  _full((H, 1)), _full((1, 1))],
        out_specs=pl.BlockSpec((1024, 1), lambda i: (i, 0)),
        out_shape=sds((ELPAD, 1), f32),
    )(He, p['dec2']['W'], p['dec2']['b'].reshape(1, 1))

    pred = predp[:EL_N, 0]
    return pred, alpha_ui[:E_N], alpha_iu[:E_N]


# speedup vs baseline: 1.3651x; 1.0417x over previous
import jax
import jax.numpy as jnp
from jax import lax
from jax.experimental import pallas as pl
from jax.experimental.pallas import tpu as pltpu
from jax.experimental.pallas import tpu_sc as plsc

H = 128
HH = 64
N = 50000
E_N = 300000
EL_N = 200000
EPAD = 327680
ELPAD = 204800
NPAD = 50176
FCH = 5120
FAGG = 51200
FTR = FCH // 16
TB = 1000

f32 = jnp.float32
i32 = jnp.int32


def _dot(a, b):
    return jnp.dot(a, b, preferred_element_type=f32)


def _pre1_body(xu, xi, Wsui, aui, Wdui, adui, Wsiu, aiu, Wdiu, adiu,
               Wlu, blu, Wli, bli,
               hsu_o, su_o, diu_o, hsi_o, si_o, dui_o,
               linu_o, lini_o):
    xu_ = xu[:]
    xi_ = xi[:]
    hsu = _dot(xu_, Wsui[:])
    hsu_o[:] = hsu
    su_o[:] = _dot(hsu, aui[:])
    diu_o[:] = _dot(_dot(xu_, Wdiu[:]), adiu[:])
    hsi = _dot(xi_, Wsiu[:])
    hsi_o[:] = hsi
    si_o[:] = _dot(hsi, aiu[:])
    dui_o[:] = _dot(_dot(xi_, Wdui[:]), adui[:])
    linu_o[:] = _dot(xu_, Wlu[:]) + blu[:]
    lini_o[:] = _dot(xi_, Wli[:]) + bli[:]


def _mid_body(aggi, aggu, lini, linu, b1ui, b1iu,
              Ws2ui, as2ui, Wd2ui, ad2ui, Ws2iu, as2iu, Wd2iu, ad2iu,
              hs2u_o, s2u_o, d2iu_o, hs2i_o, s2i_o, d2ui_o):
    hi = jnp.maximum(aggi[:] + b1ui[:] + lini[:], 0.0)
    hu = jnp.maximum(aggu[:] + b1iu[:] + linu[:], 0.0)
    hs2u = _dot(hu, Ws2ui[:])
    hs2u_o[:] = hs2u
    s2u_o[:] = _dot(hs2u, as2ui[:])
    d2iu_o[:] = _dot(_dot(hu, Wd2iu[:]), ad2iu[:])
    hs2i = _dot(hi, Ws2iu[:])
    hs2i_o[:] = hs2i
    s2i_o[:] = _dot(hs2i, as2iu[:])
    d2ui_o[:] = _dot(_dot(hi, Wd2ui[:]), ad2ui[:])


def _decpre_body(aggzu, aggzi, b2iu, b2ui, Wtop, Wbot, b1d, Zu_o, Zi_o):
    Zu_o[:] = _dot(aggzu[:] + b2iu[:], Wtop[:]) + b1d[:]
    Zi_o[:] = _dot(aggzi[:] + b2ui[:], Wbot[:])


def _decpost_body(He, w2, b2, out_o):
    h = jnp.maximum(He[:], 0.0)
    z = _dot(h, w2[:]) + b2[:]
    out_o[:] = jax.nn.sigmoid(z)


def _full(shape):
    return pl.BlockSpec(shape, lambda i: (0, 0))


def _rows(width):
    return pl.BlockSpec((TB, width), lambda i: (i, 0))


def _edge2_body(s0_h, d0_h, row0_h, col0_h,
                s1_h, d1_h, row1_h, col1_h,
                eraw0_h, eraw1_h, den_h,
                s_v, d_v, row_v, col_v, col2d_v, eraw_v, zbuf,
                den_sp0, den_sp1):
    cid = lax.axis_index("c")
    sid = lax.axis_index("s")
    wid = sid * 2 + cid

    def zb(k, c):
        zbuf[pl.ds(k * 16, 16)] = jnp.zeros((16,), f32)
        return c
    lax.fori_loop(0, NPAD // 16 // 16, zb, 0)
    dslc = pl.ds(pl.multiple_of(sid * (NPAD // 16), 8), NPAD // 16)
    pltpu.sync_copy(zbuf, den_sp0.at[dslc])
    pltpu.sync_copy(zbuf, den_sp1.at[dslc])
    plsc.subcore_barrier()

    ebase = wid * (EPAD // 32)
    for q, (s_h, d_h, row_h, col_h, eraw_h, den_sp) in enumerate([
            (s0_h, d0_h, row0_h, col0_h, eraw0_h, den_sp0),
            (s1_h, d1_h, row1_h, col1_h, eraw1_h, den_sp1)]):
        pltpu.sync_copy(s_h, s_v)
        pltpu.sync_copy(d_h, d_v)

        def chbody(ch, c0):
            cbase = pl.multiple_of(ebase + ch * 2048, 2048)
            pltpu.sync_copy(row_h.at[pl.ds(cbase, 2048)], row_v)
            pltpu.sync_copy(col_h.at[pl.ds(cbase, 2048)], col_v)

            def body(g, c):
                sl = pl.ds(g * 16, 16)
                rv = row_v[sl]
                cv = col_v[sl]
                sv = plsc.load_gather(s_v, [rv])
                dv = plsc.load_gather(d_v, [cv])
                a = sv + dv
                a = jnp.where(a > 0, a, 0.2 * a)
                e = jnp.exp(a)
                eid = cbase + g * 16 + lax.iota(i32, 16)
                e = jnp.where(eid < E_N, e, 0.0)
                eraw_v[sl] = e
                col2d_v[g // 8, pl.ds((g % 8) * 16, 16)] = cv
                return c
            lax.fori_loop(0, 128, body, 0)
            pltpu.sync_copy(eraw_v, eraw_h.at[pl.ds(cbase, 2048)])

            def kbody(k, c2):
                pltpu.sync_copy(
                    eraw_v.at[pl.ds(pl.multiple_of(k * 128, 128), 128)],
                    den_sp.at[col2d_v.at[k]], add=True)
                return c2
            lax.fori_loop(0, 16, kbody, 0)
            return c0
        lax.fori_loop(0, 5, chbody, 0)
    plsc.subcore_barrier()
    for q, den_sp in enumerate([den_sp0, den_sp1]):
        doff = pl.multiple_of((cid * 2 + q) * NPAD + sid * (NPAD // 16), 8)
        pltpu.sync_copy(den_sp.at[dslc], zbuf)
        pltpu.sync_copy(zbuf, den_h.at[pl.ds(doff, NPAD // 16)])


def _alpha2_body(row0_h, col0_h, eraw0_h, row1_h, col1_h, eraw1_h, den_h,
                 alpha0_h, alpha1_h, packed0_h, packed1_h,
                 den_v, den2_v, pack_v, e_v, a_v):
    cid = lax.axis_index("c")
    sid = lax.axis_index("s")
    wid = sid * 2 + cid
    ebase = wid * (EPAD // 32)
    for q, (row_h, col_h, eraw_h, alpha_h, packed_h) in enumerate([
            (row0_h, col0_h, eraw0_h, alpha0_h, packed0_h),
            (row1_h, col1_h, eraw1_h, alpha1_h, packed1_h)]):
        pltpu.sync_copy(den_h.at[pl.ds(q * NPAD, NPAD)], den_v)
        pltpu.sync_copy(den_h.at[pl.ds((2 + q) * NPAD, NPAD)], den2_v)

        def addb(k, c):
            sl = pl.ds(k * 16, 16)
            den_v[sl] = den_v[sl] + den2_v[sl]
            return c
        lax.fori_loop(0, NPAD // 16, addb, 0)

        def chbody(ch, c0):
            cbase = pl.multiple_of(ebase + ch * 2048, 2048)
            pltpu.sync_copy(row_h.at[pl.ds(cbase, 2048)],
                            pack_v.at[pl.ds(0, 2048)])
            pltpu.sync_copy(col_h.at[pl.ds(cbase, 2048)],
                            pack_v.at[pl.ds(2048, 2048)])
            pltpu.sync_copy(eraw_h.at[pl.ds(cbase, 2048)], e_v)

            def body(g, c):
                sl = pl.ds(g * 16, 16)
                cv = pack_v[pl.ds(2048 + g * 16, 16)]
                ev = e_v[sl]
                dv = plsc.load_gather(den_v, [cv])
                al = ev / (dv + 1e-16)
                a_v[sl] = al
                pack_v[pl.ds(4096 + g * 16, 16)] = plsc.bitcast(al, i32)
                return c
            lax.fori_loop(0, 128, body, 0)
            pltpu.sync_copy(a_v, alpha_h.at[pl.ds(cbase, 2048)])
            pltpu.sync_copy(pack_v,
                            packed_h.at[pl.ds(pl.multiple_of(cbase * 3,
                                                             2048), 6144)])
            return c0
        lax.fori_loop(0, 5, chbody, 0)


def _feat2_body(packed0_h, hs0_h, packed1_h, hs1_h,
                agg0_h, agg1_h,
                scan_v, comp_pack, comp_al,
                idx_row, idx_dst, idx_row2, idx_dst2, grows, grows2, zbuf,
                out_sp, sem, sem2, sems, sems2):
    cid = lax.axis_index("c")
    sid = lax.axis_index("s")

    def zb(r, c):
        for k in range(8):
            zbuf[r, pl.ds(k * 16, 16)] = jnp.zeros((16,), f32)
        return c
    lax.fori_loop(0, 64, zb, 0)

    rb = pl.multiple_of(sid * FTR, 8)
    for q, (packed_h, hs_h, agg_h) in enumerate([
            (packed0_h, hs0_h, agg0_h),
            (packed1_h, hs1_h, agg1_h)]):

        def tbody(t, c9):
            ck = cid * 5 + t
            lo = pl.multiple_of(ck * FCH, 128)
            hi = lo + FCH

            for i in range(5):
                pltpu.sync_copy(
                    zbuf,
                    out_sp.at[pl.ds(pl.multiple_of(rb + i * 64, 8), 64)])
            plsc.subcore_barrier()

            sbase = sid * (EPAD // 16)

            def chbody(ch, ptr):
                cbase = pl.multiple_of(sbase + ch * 2048, 2048)
                pltpu.sync_copy(
                    packed_h.at[pl.ds(pl.multiple_of(cbase * 3, 2048),
                                      6144)], scan_v)

                def sbody(g2, ptr):
                    for g in (g2 * 2, g2 * 2 + 1):
                        rv = scan_v[pl.ds(g * 16, 16)]
                        cv = scan_v[pl.ds(2048 + g * 16, 16)]
                        av = plsc.bitcast(
                            scan_v[pl.ds(4096 + g * 16, 16)], f32)
                        m = (cv >= lo) & (cv < hi) & (av != 0.0)
                        mi = jnp.where(m, 1, 0).astype(i32)
                        pk = rv + ((cv - lo) << 16)
                        psl = pl.ds(ptr, 16)
                        plsc.store_compressed(comp_pack.at[psl], pk, mask=m)
                        plsc.store_compressed(comp_al.at[psl], av, mask=m)
                        ptr = ptr + jnp.sum(mi)
                    return ptr
                return lax.fori_loop(0, 64, sbody, ptr)
            ptr = lax.fori_loop(0, 10, chbody, jnp.int32(0))

            cntp = ((ptr + 127) // 128) * 128
            zi16 = jnp.zeros((16,), i32)
            zf16 = jnp.zeros((16,), f32)

            def pbody(i, c):
                idxs = ptr + i * 16 + lax.iota(i32, 16)
                pm = idxs < cntp
                plsc.store_scatter(comp_pack, [idxs], zi16, mask=pm)
                plsc.store_scatter(comp_al, [idxs], zf16, mask=pm)
                return c
            lax.fori_loop(0, 8, pbody, 0)

            nb = cntp // 128

            def prep(off, idxr, idxd):
                def cp(i, c2):
                    s16 = pl.ds(off + i * 16, 16)
                    d16 = pl.ds(i * 16, 16)
                    pk = comp_pack[s16]
                    idxr[d16] = pk & 0xFFFF
                    idxd[d16] = pk >> 16
                    return c2
                lax.fori_loop(0, 8, cp, 0)

            def mul(off, g):
                def mul_r(r4, c2):
                    for r2 in range(4):
                        r = r4 * 4 + r2
                        av = plsc.load_gather(
                            comp_al, [jnp.full((16,), off + r, i32)])
                        for k in range(8):
                            sl = pl.ds(k * 16, 16)
                            g[r, sl] = g[r, sl] * av
                    return c2
                lax.fori_loop(0, 32, mul_r, 0)

            @pl.when(nb > 0)
            def _():
                prep(0, idx_row, idx_dst)
                pltpu.async_copy(hs_h.at[idx_row], grows, sem)

            def pair(i, c):
                b0 = i * 2
                pltpu.make_async_copy(hs_h.at[idx_row], grows, sem).wait()

                @pl.when(b0 + 1 < nb)
                def _():
                    @pl.when(i > 0)
                    def _():
                        pltpu.make_async_copy(
                            grows2, out_sp.at[idx_dst2], sems2).wait()
                    prep((b0 + 1) * 128, idx_row2, idx_dst2)
                    pltpu.async_copy(hs_h.at[idx_row2], grows2, sem2)
                mul(b0 * 128, grows)
                pltpu.async_copy(grows, out_sp.at[idx_dst], sems, add=True)

                @pl.when(b0 + 1 < nb)
                def _():
                    pltpu.make_async_copy(hs_h.at[idx_row2], grows2,
                                          sem2).wait()

                    @pl.when(b0 + 2 < nb)
                    def _():
                        pltpu.make_async_copy(
                            grows, out_sp.at[idx_dst], sems).wait()
                        prep((b0 + 2) * 128, idx_row, idx_dst)
                        pltpu.async_copy(hs_h.at[idx_row], grows, sem)
                    mul((b0 + 1) * 128, grows2)
                    pltpu.async_copy(grows2, out_sp.at[idx_dst2], sems2,
                                     add=True)
                return c
            lax.fori_loop(0, (nb + 1) // 2, pair, 0)

            @pl.when(nb > 0)
            def _():
                pltpu.make_async_copy(grows, out_sp.at[idx_dst],
                                      sems).wait()

            @pl.when(nb > 1)
            def _():
                pltpu.make_async_copy(grows2, out_sp.at[idx_dst2],
                                      sems2).wait()
            plsc.subcore_barrier()
            for i in range(2):
                roff = pl.multiple_of(rb + i * 128, 8)
                pltpu.sync_copy(out_sp.at[pl.ds(roff, 128)], grows)
                pltpu.sync_copy(
                    grows, agg_h.at[pl.ds(pl.multiple_of(lo + roff, 8),
                                          128)])
            roff = pl.multiple_of(rb + 256, 8)
            pltpu.sync_copy(out_sp.at[pl.ds(roff, 64)],
                            grows.at[pl.ds(0, 64)])
            pltpu.sync_copy(
                grows.at[pl.ds(0, 64)],
                agg_h.at[pl.ds(pl.multiple_of(lo + roff, 8), 64)])
            plsc.subcore_barrier()
            return c9
        lax.fori_loop(0, 5, tbody, 0)


def _dec_body(rowl_h, coll_h, zu_h, zi_h, he_h,
              idx_u, idx_i, idx_u2, idx_i2, gu, gi, gu2, gi2, sem, sem2):
    cid = lax.axis_index("c")
    sid = lax.axis_index("s")
    wid = sid * 2 + cid
    base = wid * (ELPAD // 32)
    nb = ELPAD // 32 // 128

    def start(b, iu, ii, bu, bi, s):
        off = pl.multiple_of(base + b * 128, 128)
        pltpu.sync_copy(rowl_h.at[pl.ds(off, 128)], iu)
        pltpu.sync_copy(coll_h.at[pl.ds(off, 128)], ii)
        pltpu.async_copy(zu_h.at[iu], bu, s)
        pltpu.async_copy(zi_h.at[ii], bi, s)

    def finish(b, iu, ii, bu, bi, s):
        off = pl.multiple_of(base + b * 128, 128)
        pltpu.make_async_copy(zu_h.at[iu], bu, s).wait()
        pltpu.make_async_copy(zi_h.at[ii], bi, s).wait()

        def addr(r4, c2):
            for r2 in range(4):
                r = r4 * 4 + r2
                for k in range(8):
                    sl = pl.ds(k * 16, 16)
                    bu[r, sl] = bu[r, sl] + bi[r, sl]
            return c2
        lax.fori_loop(0, 32, addr, 0)
        pltpu.sync_copy(bu, he_h.at[pl.ds(off, 128)])

    start(0, idx_u, idx_i, gu, gi, sem)

    def body(p, c):
        b0 = p * 2
        start(b0 + 1, idx_u2, idx_i2, gu2, gi2, sem2)
        finish(b0, idx_u, idx_i, gu, gi, sem)

        @pl.when(b0 + 2 < nb)
        def _():
            start(b0 + 2, idx_u, idx_i, gu, gi, sem)
        finish(b0 + 1, idx_u2, idx_i2, gu2, gi2, sem2)
        return c
    lax.fori_loop(0, nb // 2, body, 0)


def _mesh():
    return plsc.VectorSubcoreMesh(core_axis_name="c", subcore_axis_name="s")


_SC_PARAMS = pltpu.CompilerParams(needs_layout_passes=False)


def _edge2_call(s0, d0, row0, col0, s1, d1, row1, col1):
    return pl.kernel(
        _edge2_body,
        out_type=[jax.ShapeDtypeStruct((EPAD,), f32),
                  jax.ShapeDtypeStruct((EPAD,), f32),
                  jax.ShapeDtypeStruct((4 * NPAD,), f32)],
        mesh=_mesh(),
        compiler_params=_SC_PARAMS,
        scratch_types=[
            pltpu.VMEM((N,), f32),
            pltpu.VMEM((N,), f32),
            pltpu.VMEM((2048,), i32),
            pltpu.VMEM((2048,), i32),
            pltpu.VMEM((16, 128), i32),
            pltpu.VMEM((2048,), f32),
            pltpu.VMEM((NPAD // 16,), f32),
            pltpu.VMEM_SHARED((NPAD,), f32),
            pltpu.VMEM_SHARED((NPAD,), f32),
        ],
    )(s0, d0, row0, col0, s1, d1, row1, col1)


def _alpha2_call(row0, col0, eraw0, row1, col1, eraw1, den4):
    return pl.kernel(
        _alpha2_body,
        out_type=[jax.ShapeDtypeStruct((EPAD,), f32),
                  jax.ShapeDtypeStruct((EPAD,), f32),
                  jax.ShapeDtypeStruct((3 * EPAD,), i32),
                  jax.ShapeDtypeStruct((3 * EPAD,), i32)],
        mesh=_mesh(),
        compiler_params=_SC_PARAMS,
        scratch_types=[
            pltpu.VMEM((NPAD,), f32),
            pltpu.VMEM((NPAD,), f32),
            pltpu.VMEM((6144,), i32),
            pltpu.VMEM((2048,), f32),
            pltpu.VMEM((2048,), f32),
        ],
    )(row0, col0, eraw0, row1, col1, eraw1, den4)


def _feat2_call(packed0, hs0, packed1, hs1):
    return pl.kernel(
        _feat2_body,
        out_type=[jax.ShapeDtypeStruct((FAGG, H), f32),
                  jax.ShapeDtypeStruct((FAGG, H), f32)],
        mesh=_mesh(),
        compiler_params=_SC_PARAMS,
        scratch_types=[
            pltpu.VMEM((6144,), i32),
            pltpu.VMEM((20608,), i32),
            pltpu.VMEM((20608,), f32),
            pltpu.VMEM((128,), i32),
            pltpu.VMEM((128,), i32),
            pltpu.VMEM((128,), i32),
            pltpu.VMEM((128,), i32),
            pltpu.VMEM((128, H), f32),
            pltpu.VMEM((128, H), f32),
            pltpu.VMEM((64, H), f32),
            pltpu.VMEM_SHARED((FCH, H), f32),
            pltpu.SemaphoreType.DMA,
            pltpu.SemaphoreType.DMA,
            pltpu.SemaphoreType.DMA,
            pltpu.SemaphoreType.DMA,
        ],
    )(packed0, hs0, packed1, hs1)


def _dec_call(rowl, coll, Zu, Zi):
    return pl.kernel(
        _dec_body,
        out_type=[jax.ShapeDtypeStruct((ELPAD, H), f32)],
        mesh=_mesh(),
        compiler_params=_SC_PARAMS,
        scratch_types=[
            pltpu.VMEM((128,), i32),
            pltpu.VMEM((128,), i32),
            pltpu.VMEM((128,), i32),
            pltpu.VMEM((128,), i32),
            pltpu.VMEM((128, H), f32),
            pltpu.VMEM((128, H), f32),
            pltpu.VMEM((128, H), f32),
            pltpu.VMEM((128, H), f32),
            pltpu.SemaphoreType.DMA,
            pltpu.SemaphoreType.DMA,
        ],
    )(rowl, coll, Zu, Zi)[0]


def _run_layer(s0, d0, hs0, s1, d1, hs1,
               row0, col0, row1, col1):
    eraw0, eraw1, den4 = _edge2_call(s0, d0, row0, col0,
                                     s1, d1, row1, col1)
    alpha0, alpha1, packed0, packed1 = _alpha2_call(
        row0, col0, eraw0, row1, col1, eraw1, den4)
    agg0, agg1 = _feat2_call(packed0, hs0, packed1, hs1)
    return agg0, agg1, alpha0, alpha1


def kernel(x_user, x_item, edge_index_ui, edge_index_iu, edge_label_index,
           params):
    p = params
    def padE(x):
        return jnp.pad(x.astype(i32), (0, EPAD - E_N), mode='wrap')

    def padL(x):
        return jnp.pad(x.astype(i32), (0, ELPAD - EL_N), mode='wrap')

    row_ui = padE(edge_index_ui[0])
    col_ui = padE(edge_index_ui[1])
    row_iu = padE(edge_index_iu[0])
    col_iu = padE(edge_index_iu[1])
    rowl = padL(edge_label_index[0])
    coll = padL(edge_label_index[1])

    c1ui, c1iu = p['conv1_ui'], p['conv1_iu']
    c2ui, c2iu = p['conv2_ui'], p['conv2_iu']
    lu, li = p['lin1_user'], p['lin1_item']

    nt = N // TB
    v = lambda x: x.reshape(H, 1)
    b = lambda x: x.reshape(1, H)
    sds = jax.ShapeDtypeStruct

    (hsu, su1, diu1, hsi, si1, dui1, linu, lini) = pl.pallas_call(
        _pre1_body,
        grid=(nt,),
        in_specs=[_rows(H), _rows(H)] + [_full((H, H)), _full((H, 1))] * 4
        + [_full((H, H)), _full((1, H))] * 2,
        out_specs=[_rows(H), _rows(1), _rows(1), _rows(H), _rows(1),
                   _rows(1), _rows(H), _rows(H)],
        out_shape=[sds((N, H), f32), sds((N, 1), f32), sds((N, 1), f32),
                   sds((N, H), f32), sds((N, 1), f32), sds((N, 1), f32),
                   sds((N, H), f32), sds((N, H), f32)],
    )(x_user, x_item,
      c1ui['Ws'], v(c1ui['as']), c1ui['Wd'], v(c1ui['ad']),
      c1iu['Ws'], v(c1iu['as']), c1iu['Wd'], v(c1iu['ad']),
      lu['W'], b(lu['b']), li['W'], b(li['b']))

    agg_i1, agg_u1, _, _ = _run_layer(
        su1.reshape(-1), dui1.reshape(-1), hsu,
        si1.reshape(-1), diu1.reshape(-1), hsi,
        row_ui, col_ui, row_iu, col_iu)

    (hs2u, s2u, d2iu, hs2i, s2i, d2ui) = pl.pallas_call(
        _mid_body,
        grid=(nt,),
        in_specs=[_rows(H)] * 4 + [_full((1, H))] * 2
        + [_full((H, H)), _full((H, 1))] * 4,
        out_specs=[_rows(H), _rows(1), _rows(1), _rows(H), _rows(1),
                   _rows(1)],
        out_shape=[sds((N, H), f32), sds((N, 1), f32), sds((N, 1), f32),
                   sds((N, H), f32), sds((N, 1), f32), sds((N, 1), f32)],
    )(agg_i1[:N], agg_u1[:N], lini, linu, b(c1ui['b']), b(c1iu['b']),
      c2ui['Ws'], v(c2ui['as']), c2ui['Wd'], v(c2ui['ad']),
      c2iu['Ws'], v(c2iu['as']), c2iu['Wd'], v(c2iu['ad']))

    agg_zi, agg_zu, alpha_ui, alpha_iu = _run_layer(
        s2u.reshape(-1), d2ui.reshape(-1), hs2u,
        s2i.reshape(-1), d2iu.reshape(-1), hs2i,
        row_ui, col_ui, row_iu, col_iu)

    Wd1 = p['dec1']['W']
    Zu, Zi = pl.pallas_call(
        _decpre_body,
        grid=(nt,),
        in_specs=[_rows(H), _rows(H), _full((1, H)), _full((1, H)),
                  _full((H, H)), _full((H, H)), _full((1, H))],
        out_specs=[_rows(H), _rows(H)],
        out_shape=[sds((N, H), f32), sds((N, H), f32)],
    )(agg_zu[:N], agg_zi[:N], b(c2iu['b']), b(c2ui['b']),
      Wd1[:H], Wd1[H:], b(p['dec1']['b']))

    He = _dec_call(rowl, coll, Zu, Zi)

    predp = pl.pallas_call(
        _decpost_body,
        grid=(ELPAD // 1024,),
        in_specs=[pl.BlockSpec((1024, H), lambda i: (i, 0)),
                  _full((H, 1)), _full((1, 1))],
        out_specs=pl.BlockSpec((1024, 1), lambda i: (i, 0)),
        out_shape=sds((ELPAD, 1), f32),
    )(He, p['dec2']['W'], p['dec2']['b'].reshape(1, 1))

    pred = predp[:EL_N, 0]
    return pred, alpha_ui[:E_N], alpha_iu[:E_N]


# speedup vs baseline: 1.3715x; 1.0046x over previous
import jax
import jax.numpy as jnp
from jax import lax
from jax.experimental import pallas as pl
from jax.experimental.pallas import tpu as pltpu
from jax.experimental.pallas import tpu_sc as plsc

H = 128
HH = 64
N = 50000
E_N = 300000
EL_N = 200000
EPAD = 327680
ELPAD = 204800
NPAD = 50176
FCH = 5120
FAGG = 51200
FTR = FCH // 16
TB = 1000

f32 = jnp.float32
i32 = jnp.int32


def _dot(a, b):
    return jnp.dot(a, b, preferred_element_type=f32)


def _pre1_body(xu, xi, Wsui, aui, Wdui, adui, Wsiu, aiu, Wdiu, adiu,
               Wlu, blu, Wli, bli,
               hsu_o, su_o, diu_o, hsi_o, si_o, dui_o,
               linu_o, lini_o):
    xu_ = xu[:]
    xi_ = xi[:]
    hsu = _dot(xu_, Wsui[:])
    hsu_o[:] = hsu
    su_o[:] = _dot(hsu, aui[:])
    diu_o[:] = _dot(_dot(xu_, Wdiu[:]), adiu[:])
    hsi = _dot(xi_, Wsiu[:])
    hsi_o[:] = hsi
    si_o[:] = _dot(hsi, aiu[:])
    dui_o[:] = _dot(_dot(xi_, Wdui[:]), adui[:])
    linu_o[:] = _dot(xu_, Wlu[:]) + blu[:]
    lini_o[:] = _dot(xi_, Wli[:]) + bli[:]


def _mid_body(aggi, aggu, lini, linu, b1ui, b1iu,
              Ws2ui, as2ui, Wd2ui, ad2ui, Ws2iu, as2iu, Wd2iu, ad2iu,
              hs2u_o, s2u_o, d2iu_o, hs2i_o, s2i_o, d2ui_o):
    hi = jnp.maximum(aggi[:] + b1ui[:] + lini[:], 0.0)
    hu = jnp.maximum(aggu[:] + b1iu[:] + linu[:], 0.0)
    hs2u = _dot(hu, Ws2ui[:])
    hs2u_o[:] = hs2u
    s2u_o[:] = _dot(hs2u, as2ui[:])
    d2iu_o[:] = _dot(_dot(hu, Wd2iu[:]), ad2iu[:])
    hs2i = _dot(hi, Ws2iu[:])
    hs2i_o[:] = hs2i
    s2i_o[:] = _dot(hs2i, as2iu[:])
    d2ui_o[:] = _dot(_dot(hi, Wd2ui[:]), ad2ui[:])


def _decpre_body(aggzu, aggzi, b2iu, b2ui, Wtop, Wbot, b1d, Zu_o, Zi_o):
    Zu_o[:] = _dot(aggzu[:] + b2iu[:], Wtop[:]) + b1d[:]
    Zi_o[:] = _dot(aggzi[:] + b2ui[:], Wbot[:])


def _decpost_body(He, w2, b2, out_o):
    h = jnp.maximum(He[:], 0.0)
    z = _dot(h, w2[:]) + b2[:]
    out_o[:] = jax.nn.sigmoid(z)


def _full(shape):
    return pl.BlockSpec(shape, lambda i: (0, 0))


def _rows(width):
    return pl.BlockSpec((TB, width), lambda i: (i, 0))


def _edge2_body(s0_h, d0_h, row0_h, col0_h,
                s1_h, d1_h, row1_h, col1_h,
                eraw0_h, eraw1_h, den_h,
                s_v, d_v, row_v, col_v, col2d_v, eraw_v, zbuf,
                den_sp0, den_sp1, semk):
    cid = lax.axis_index("c")
    sid = lax.axis_index("s")
    wid = sid * 2 + cid

    def zb(k, c):
        zbuf[pl.ds(k * 16, 16)] = jnp.zeros((16,), f32)
        return c
    lax.fori_loop(0, NPAD // 16 // 16, zb, 0)
    dslc = pl.ds(pl.multiple_of(sid * (NPAD // 16), 8), NPAD // 16)
    pltpu.sync_copy(zbuf, den_sp0.at[dslc])
    pltpu.sync_copy(zbuf, den_sp1.at[dslc])
    plsc.subcore_barrier()

    ebase = wid * (EPAD // 32)
    for q, (s_h, d_h, row_h, col_h, eraw_h, den_sp) in enumerate([
            (s0_h, d0_h, row0_h, col0_h, eraw0_h, den_sp0),
            (s1_h, d1_h, row1_h, col1_h, eraw1_h, den_sp1)]):
        pltpu.sync_copy(s_h, s_v)
        pltpu.sync_copy(d_h, d_v)

        def chbody(ch, c0):
            cbase = pl.multiple_of(ebase + ch * 2048, 2048)
            pltpu.sync_copy(row_h.at[pl.ds(cbase, 2048)], row_v)
            pltpu.sync_copy(col_h.at[pl.ds(cbase, 2048)], col_v)

            def body(g, c):
                sl = pl.ds(g * 16, 16)
                rv = row_v[sl]
                cv = col_v[sl]
                sv = plsc.load_gather(s_v, [rv])
                dv = plsc.load_gather(d_v, [cv])
                a = sv + dv
                a = jnp.where(a > 0, a, 0.2 * a)
                e = jnp.exp(a)
                eid = cbase + g * 16 + lax.iota(i32, 16)
                e = jnp.where(eid < E_N, e, 0.0)
                eraw_v[sl] = e
                col2d_v[g // 8, pl.ds((g % 8) * 16, 16)] = cv
                return c
            lax.fori_loop(0, 128, body, 0)
            pltpu.sync_copy(eraw_v, eraw_h.at[pl.ds(cbase, 2048)])

            def kbody(k, c2):
                pltpu.async_copy(
                    eraw_v.at[pl.ds(pl.multiple_of(k * 128, 128), 128)],
                    den_sp.at[col2d_v.at[k]], semk, add=True)
                return c2
            lax.fori_loop(0, 16, kbody, 0)

            def kdrain(k, c2):
                pltpu.make_async_copy(
                    eraw_v.at[pl.ds(pl.multiple_of(k * 128, 128), 128)],
                    den_sp.at[col2d_v.at[k]], semk).wait()
                return c2
            lax.fori_loop(0, 16, kdrain, 0)
            return c0
        lax.fori_loop(0, 5, chbody, 0)
    plsc.subcore_barrier()
    for q, den_sp in enumerate([den_sp0, den_sp1]):
        doff = pl.multiple_of((cid * 2 + q) * NPAD + sid * (NPAD // 16), 8)
        pltpu.sync_copy(den_sp.at[dslc], zbuf)
        pltpu.sync_copy(zbuf, den_h.at[pl.ds(doff, NPAD // 16)])


def _alpha2_body(row0_h, col0_h, eraw0_h, row1_h, col1_h, eraw1_h, den_h,
                 alpha0_h, alpha1_h, packed0_h, packed1_h,
                 den_v, den2_v, pack_v, e_v, a_v):
    cid = lax.axis_index("c")
    sid = lax.axis_index("s")
    wid = sid * 2 + cid
    ebase = wid * (EPAD // 32)
    for q, (row_h, col_h, eraw_h, alpha_h, packed_h) in enumerate([
            (row0_h, col0_h, eraw0_h, alpha0_h, packed0_h),
            (row1_h, col1_h, eraw1_h, alpha1_h, packed1_h)]):
        pltpu.sync_copy(den_h.at[pl.ds(q * NPAD, NPAD)], den_v)
        pltpu.sync_copy(den_h.at[pl.ds((2 + q) * NPAD, NPAD)], den2_v)

        def addb(k, c):
            sl = pl.ds(k * 16, 16)
            den_v[sl] = den_v[sl] + den2_v[sl]
            return c
        lax.fori_loop(0, NPAD // 16, addb, 0)

        def chbody(ch, c0):
            cbase = pl.multiple_of(ebase + ch * 2048, 2048)
            pltpu.sync_copy(row_h.at[pl.ds(cbase, 2048)],
                            pack_v.at[pl.ds(0, 2048)])
            pltpu.sync_copy(col_h.at[pl.ds(cbase, 2048)],
                            pack_v.at[pl.ds(2048, 2048)])
            pltpu.sync_copy(eraw_h.at[pl.ds(cbase, 2048)], e_v)

            def body(g, c):
                sl = pl.ds(g * 16, 16)
                cv = pack_v[pl.ds(2048 + g * 16, 16)]
                ev = e_v[sl]
                dv = plsc.load_gather(den_v, [cv])
                al = ev / (dv + 1e-16)
                a_v[sl] = al
                pack_v[pl.ds(4096 + g * 16, 16)] = plsc.bitcast(al, i32)
                return c
            lax.fori_loop(0, 128, body, 0)
            pltpu.sync_copy(a_v, alpha_h.at[pl.ds(cbase, 2048)])
            pltpu.sync_copy(pack_v,
                            packed_h.at[pl.ds(pl.multiple_of(cbase * 3,
                                                             2048), 6144)])
            return c0
        lax.fori_loop(0, 5, chbody, 0)


def _feat2_body(packed0_h, hs0_h, packed1_h, hs1_h,
                agg0_h, agg1_h,
                scan_v, comp_pack, comp_al,
                idx_row, idx_dst, idx_row2, idx_dst2, grows, grows2, zbuf,
                out_sp, sem, sem2, sems, sems2):
    cid = lax.axis_index("c")
    sid = lax.axis_index("s")

    def zb(r, c):
        for k in range(8):
            zbuf[r, pl.ds(k * 16, 16)] = jnp.zeros((16,), f32)
        return c
    lax.fori_loop(0, 64, zb, 0)

    rb = pl.multiple_of(sid * FTR, 8)
    for q, (packed_h, hs_h, agg_h) in enumerate([
            (packed0_h, hs0_h, agg0_h),
            (packed1_h, hs1_h, agg1_h)]):

        def tbody(t, c9):
            ck = cid * 5 + t
            lo = pl.multiple_of(ck * FCH, 128)
            hi = lo + FCH

            for i in range(5):
                pltpu.sync_copy(
                    zbuf,
                    out_sp.at[pl.ds(pl.multiple_of(rb + i * 64, 8), 64)])
            plsc.subcore_barrier()

            sbase = sid * (EPAD // 16)

            def chbody(ch, ptr):
                cbase = pl.multiple_of(sbase + ch * 2048, 2048)
                pltpu.sync_copy(
                    packed_h.at[pl.ds(pl.multiple_of(cbase * 3, 2048),
                                      6144)], scan_v)

                def sbody(g2, ptr):
                    for g in (g2 * 2, g2 * 2 + 1):
                        rv = scan_v[pl.ds(g * 16, 16)]
                        cv = scan_v[pl.ds(2048 + g * 16, 16)]
                        av = plsc.bitcast(
                            scan_v[pl.ds(4096 + g * 16, 16)], f32)
                        m = (cv >= lo) & (cv < hi) & (av != 0.0)
                        mi = jnp.where(m, 1, 0).astype(i32)
                        pk = rv + ((cv - lo) << 16)
                        psl = pl.ds(ptr, 16)
                        plsc.store_compressed(comp_pack.at[psl], pk, mask=m)
                        plsc.store_compressed(comp_al.at[psl], av, mask=m)
                        ptr = ptr + jnp.sum(mi)
                    return ptr
                return lax.fori_loop(0, 64, sbody, ptr)
            ptr = lax.fori_loop(0, 10, chbody, jnp.int32(0))

            cntp = ((ptr + 127) // 128) * 128
            zi16 = jnp.zeros((16,), i32)
            zf16 = jnp.zeros((16,), f32)

            def pbody(i, c):
                idxs = ptr + i * 16 + lax.iota(i32, 16)
                pm = idxs < cntp
                plsc.store_scatter(comp_pack, [idxs], zi16, mask=pm)
                plsc.store_scatter(comp_al, [idxs], zf16, mask=pm)
                return c
            lax.fori_loop(0, 8, pbody, 0)

            nb = cntp // 128

            def prep(off, idxr, idxd):
                def cp(i, c2):
                    s16 = pl.ds(off + i * 16, 16)
                    d16 = pl.ds(i * 16, 16)
                    pk = comp_pack[s16]
                    idxr[d16] = pk & 0xFFFF
                    idxd[d16] = pk >> 16
                    return c2
                lax.fori_loop(0, 8, cp, 0)

            def mul(off, g):
                def mul_r(r4, c2):
                    for r2 in range(4):
                        r = r4 * 4 + r2
                        av = plsc.load_gather(
                            comp_al, [jnp.full((16,), off + r, i32)])
                        for k in range(8):
                            sl = pl.ds(k * 16, 16)
                            g[r, sl] = g[r, sl] * av
                    return c2
                lax.fori_loop(0, 32, mul_r, 0)

            @pl.when(nb > 0)
            def _():
                prep(0, idx_row, idx_dst)
                pltpu.async_copy(hs_h.at[idx_row], grows, sem)

            def pair(i, c):
                b0 = i * 2
                pltpu.make_async_copy(hs_h.at[idx_row], grows, sem).wait()

                @pl.when(b0 + 1 < nb)
                def _():
                    @pl.when(i > 0)
                    def _():
                        pltpu.make_async_copy(
                            grows2, out_sp.at[idx_dst2], sems2).wait()
                    prep((b0 + 1) * 128, idx_row2, idx_dst2)
                    pltpu.async_copy(hs_h.at[idx_row2], grows2, sem2)
                mul(b0 * 128, grows)
                pltpu.async_copy(grows, out_sp.at[idx_dst], sems, add=True)

                @pl.when(b0 + 1 < nb)
                def _():
                    pltpu.make_async_copy(hs_h.at[idx_row2], grows2,
                                          sem2).wait()

                    @pl.when(b0 + 2 < nb)
                    def _():
                        pltpu.make_async_copy(
                            grows, out_sp.at[idx_dst], sems).wait()
                        prep((b0 + 2) * 128, idx_row, idx_dst)
                        pltpu.async_copy(hs_h.at[idx_row], grows, sem)
                    mul((b0 + 1) * 128, grows2)
                    pltpu.async_copy(grows2, out_sp.at[idx_dst2], sems2,
                                     add=True)
                return c
            lax.fori_loop(0, (nb + 1) // 2, pair, 0)

            @pl.when(nb > 0)
            def _():
                pltpu.make_async_copy(grows, out_sp.at[idx_dst],
                                      sems).wait()

            @pl.when(nb > 1)
            def _():
                pltpu.make_async_copy(grows2, out_sp.at[idx_dst2],
                                      sems2).wait()
            plsc.subcore_barrier()
            for i in range(2):
                roff = pl.multiple_of(rb + i * 128, 8)
                pltpu.sync_copy(out_sp.at[pl.ds(roff, 128)], grows)
                pltpu.sync_copy(
                    grows, agg_h.at[pl.ds(pl.multiple_of(lo + roff, 8),
                                          128)])
            roff = pl.multiple_of(rb + 256, 8)
            pltpu.sync_copy(out_sp.at[pl.ds(roff, 64)],
                            grows.at[pl.ds(0, 64)])
            pltpu.sync_copy(
                grows.at[pl.ds(0, 64)],
                agg_h.at[pl.ds(pl.multiple_of(lo + roff, 8), 64)])
            plsc.subcore_barrier()
            return c9
        lax.fori_loop(0, 5, tbody, 0)


def _dec_body(rowl_h, coll_h, zu_h, zi_h, he_h,
              idx_u, idx_i, idx_u2, idx_i2, gu, gi, gu2, gi2, sem, sem2):
    cid = lax.axis_index("c")
    sid = lax.axis_index("s")
    wid = sid * 2 + cid
    base = wid * (ELPAD // 32)
    nb = ELPAD // 32 // 128

    def start(b, iu, ii, bu, bi, s):
        off = pl.multiple_of(base + b * 128, 128)
        pltpu.sync_copy(rowl_h.at[pl.ds(off, 128)], iu)
        pltpu.sync_copy(coll_h.at[pl.ds(off, 128)], ii)
        pltpu.async_copy(zu_h.at[iu], bu, s)
        pltpu.async_copy(zi_h.at[ii], bi, s)

    def finish(b, iu, ii, bu, bi, s):
        off = pl.multiple_of(base + b * 128, 128)
        pltpu.make_async_copy(zu_h.at[iu], bu, s).wait()
        pltpu.make_async_copy(zi_h.at[ii], bi, s).wait()

        def addr(r4, c2):
            for r2 in range(4):
                r = r4 * 4 + r2
                for k in range(8):
                    sl = pl.ds(k * 16, 16)
                    bu[r, sl] = bu[r, sl] + bi[r, sl]
            return c2
        lax.fori_loop(0, 32, addr, 0)
        pltpu.sync_copy(bu, he_h.at[pl.ds(off, 128)])

    start(0, idx_u, idx_i, gu, gi, sem)

    def body(p, c):
        b0 = p * 2
        start(b0 + 1, idx_u2, idx_i2, gu2, gi2, sem2)
        finish(b0, idx_u, idx_i, gu, gi, sem)

        @pl.when(b0 + 2 < nb)
        def _():
            start(b0 + 2, idx_u, idx_i, gu, gi, sem)
        finish(b0 + 1, idx_u2, idx_i2, gu2, gi2, sem2)
        return c
    lax.fori_loop(0, nb // 2, body, 0)


def _mesh():
    return plsc.VectorSubcoreMesh(core_axis_name="c", subcore_axis_name="s")


_SC_PARAMS = pltpu.CompilerParams(needs_layout_passes=False)


def _edge2_call(s0, d0, row0, col0, s1, d1, row1, col1):
    return pl.kernel(
        _edge2_body,
        out_type=[jax.ShapeDtypeStruct((EPAD,), f32),
                  jax.ShapeDtypeStruct((EPAD,), f32),
                  jax.ShapeDtypeStruct((4 * NPAD,), f32)],
        mesh=_mesh(),
        compiler_params=_SC_PARAMS,
        scratch_types=[
            pltpu.VMEM((N,), f32),
            pltpu.VMEM((N,), f32),
            pltpu.VMEM((2048,), i32),
            pltpu.VMEM((2048,), i32),
            pltpu.VMEM((16, 128), i32),
            pltpu.VMEM((2048,), f32),
            pltpu.VMEM((NPAD // 16,), f32),
            pltpu.VMEM_SHARED((NPAD,), f32),
            pltpu.VMEM_SHARED((NPAD,), f32),
            pltpu.SemaphoreType.DMA,
        ],
    )(s0, d0, row0, col0, s1, d1, row1, col1)


def _alpha2_call(row0, col0, eraw0, row1, col1, eraw1, den4):
    return pl.kernel(
        _alpha2_body,
        out_type=[jax.ShapeDtypeStruct((EPAD,), f32),
                  jax.ShapeDtypeStruct((EPAD,), f32),
                  jax.ShapeDtypeStruct((3 * EPAD,), i32),
                  jax.ShapeDtypeStruct((3 * EPAD,), i32)],
        mesh=_mesh(),
        compiler_params=_SC_PARAMS,
        scratch_types=[
            pltpu.VMEM((NPAD,), f32),
            pltpu.VMEM((NPAD,), f32),
            pltpu.VMEM((6144,), i32),
            pltpu.VMEM((2048,), f32),
            pltpu.VMEM((2048,), f32),
        ],
    )(row0, col0, eraw0, row1, col1, eraw1, den4)


def _feat2_call(packed0, hs0, packed1, hs1):
    return pl.kernel(
        _feat2_body,
        out_type=[jax.ShapeDtypeStruct((FAGG, H), f32),
                  jax.ShapeDtypeStruct((FAGG, H), f32)],
        mesh=_mesh(),
        compiler_params=_SC_PARAMS,
        scratch_types=[
            pltpu.VMEM((6144,), i32),
            pltpu.VMEM((20608,), i32),
            pltpu.VMEM((20608,), f32),
            pltpu.VMEM((128,), i32),
            pltpu.VMEM((128,), i32),
            pltpu.VMEM((128,), i32),
            pltpu.VMEM((128,), i32),
            pltpu.VMEM((128, H), f32),
            pltpu.VMEM((128, H), f32),
            pltpu.VMEM((64, H), f32),
            pltpu.VMEM_SHARED((FCH, H), f32),
            pltpu.SemaphoreType.DMA,
            pltpu.SemaphoreType.DMA,
            pltpu.SemaphoreType.DMA,
            pltpu.SemaphoreType.DMA,
        ],
    )(packed0, hs0, packed1, hs1)


def _dec_call(rowl, coll, Zu, Zi):
    return pl.kernel(
        _dec_body,
        out_type=[jax.ShapeDtypeStruct((ELPAD, H), f32)],
        mesh=_mesh(),
        compiler_params=_SC_PARAMS,
        scratch_types=[
            pltpu.VMEM((128,), i32),
            pltpu.VMEM((128,), i32),
            pltpu.VMEM((128,), i32),
            pltpu.VMEM((128,), i32),
            pltpu.VMEM((128, H), f32),
            pltpu.VMEM((128, H), f32),
            pltpu.VMEM((128, H), f32),
            pltpu.VMEM((128, H), f32),
            pltpu.SemaphoreType.DMA,
            pltpu.SemaphoreType.DMA,
        ],
    )(rowl, coll, Zu, Zi)[0]


def _run_layer(s0, d0, hs0, s1, d1, hs1,
               row0, col0, row1, col1):
    eraw0, eraw1, den4 = _edge2_call(s0, d0, row0, col0,
                                     s1, d1, row1, col1)
    alpha0, alpha1, packed0, packed1 = _alpha2_call(
        row0, col0, eraw0, row1, col1, eraw1, den4)
    agg0, agg1 = _feat2_call(packed0, hs0, packed1, hs1)
    return agg0, agg1, alpha0, alpha1


def kernel(x_user, x_item, edge_index_ui, edge_index_iu, edge_label_index,
           params):
    p = params
    def padE(x):
        return jnp.pad(x.astype(i32), (0, EPAD - E_N), mode='wrap')

    def padL(x):
        return jnp.pad(x.astype(i32), (0, ELPAD - EL_N), mode='wrap')

    row_ui = padE(edge_index_ui[0])
    col_ui = padE(edge_index_ui[1])
    row_iu = padE(edge_index_iu[0])
    col_iu = padE(edge_index_iu[1])
    rowl = padL(edge_label_index[0])
    coll = padL(edge_label_index[1])

    c1ui, c1iu = p['conv1_ui'], p['conv1_iu']
    c2ui, c2iu = p['conv2_ui'], p['conv2_iu']
    lu, li = p['lin1_user'], p['lin1_item']

    nt = N // TB
    v = lambda x: x.reshape(H, 1)
    b = lambda x: x.reshape(1, H)
    sds = jax.ShapeDtypeStruct

    (hsu, su1, diu1, hsi, si1, dui1, linu, lini) = pl.pallas_call(
        _pre1_body,
        grid=(nt,),
        in_specs=[_rows(H), _rows(H)] + [_full((H, H)), _full((H, 1))] * 4
        + [_full((H, H)), _full((1, H))] * 2,
        out_specs=[_rows(H), _rows(1), _rows(1), _rows(H), _rows(1),
                   _rows(1), _rows(H), _rows(H)],
        out_shape=[sds((N, H), f32), sds((N, 1), f32), sds((N, 1), f32),
                   sds((N, H), f32), sds((N, 1), f32), sds((N, 1), f32),
                   sds((N, H), f32), sds((N, H), f32)],
    )(x_user, x_item,
      c1ui['Ws'], v(c1ui['as']), c1ui['Wd'], v(c1ui['ad']),
      c1iu['Ws'], v(c1iu['as']), c1iu['Wd'], v(c1iu['ad']),
      lu['W'], b(lu['b']), li['W'], b(li['b']))

    agg_i1, agg_u1, _, _ = _run_layer(
        su1.reshape(-1), dui1.reshape(-1), hsu,
        si1.reshape(-1), diu1.reshape(-1), hsi,
        row_ui, col_ui, row_iu, col_iu)

    (hs2u, s2u, d2iu, hs2i, s2i, d2ui) = pl.pallas_call(
        _mid_body,
        grid=(nt,),
        in_specs=[_rows(H)] * 4 + [_full((1, H))] * 2
        + [_full((H, H)), _full((H, 1))] * 4,
        out_specs=[_rows(H), _rows(1), _rows(1), _rows(H), _rows(1),
                   _rows(1)],
        out_shape=[sds((N, H), f32), sds((N, 1), f32), sds((N, 1), f32),
                   sds((N, H), f32), sds((N, 1), f32), sds((N, 1), f32)],
    )(agg_i1[:N], agg_u1[:N], lini, linu, b(c1ui['b']), b(c1iu['b']),
      c2ui['Ws'], v(c2ui['as']), c2ui['Wd'], v(c2ui['ad']),
      c2iu['Ws'], v(c2iu['as']), c2iu['Wd'], v(c2iu['ad']))

    agg_zi, agg_zu, alpha_ui, alpha_iu = _run_layer(
        s2u.reshape(-1), d2ui.reshape(-1), hs2u,
        s2i.reshape(-1), d2iu.reshape(-1), hs2i,
        row_ui, col_ui, row_iu, col_iu)

    Wd1 = p['dec1']['W']
    Zu, Zi = pl.pallas_call(
        _decpre_body,
        grid=(nt,),
        in_specs=[_rows(H), _rows(H), _full((1, H)), _full((1, H)),
                  _full((H, H)), _full((H, H)), _full((1, H))],
        out_specs=[_rows(H), _rows(H)],
        out_shape=[sds((N, H), f32), sds((N, H), f32)],
    )(agg_zu[:N], agg_zi[:N], b(c2iu['b']), b(c2ui['b']),
      Wd1[:H], Wd1[H:], b(p['dec1']['b']))

    He = _dec_call(rowl, coll, Zu, Zi)

    predp = pl.pallas_call(
        _decpost_body,
        grid=(ELPAD // 1024,),
        in_specs=[pl.BlockSpec((1024, H), lambda i: (i, 0)),
                  _full((H, 1)), _full((1, 1))],
        out_specs=pl.BlockSpec((1024, 1), lambda i: (i, 0)),
        out_shape=sds((ELPAD, 1), f32),
    )(He, p['dec2']['W'], p['dec2']['b'].reshape(1, 1))

    pred = predp[:EL_N, 0]
    return pred, alpha_ui[:E_N], alpha_iu[:E_N]
